# Initial kernel scaffold; baseline (speedup 1.0000x reference)
import jax
import jax.numpy as jnp
from jax.experimental import pallas as pl


def kernel(x_lidar, x_radar1, x_radar2, ei_ll, ei_r1r1, ei_r2r2, ei_lr1, ei_lr2, ea_ll, ea_r1r1, ea_r2r2, ea_lr1, ea_lr2, params):
    raise NotImplementedError("write your pallas kernel here")


# speedup vs baseline: 1.0012x; 1.0012x over previous
import functools
import jax
import jax.numpy as jnp
from jax import lax
from jax.experimental import pallas as pl
from jax.experimental.pallas import tpu as pltpu

N_L = 10000
E = 160000
HEADS = 8
HD = 32
HID = 256


def _mm_body(x_ref, w_ref, b_ref, o_ref, *, act):
    y = jnp.dot(x_ref[...], w_ref[...], preferred_element_type=jnp.float32)
    y = y + b_ref[...]
    if act == "elu":
        y = jnp.where(y > 0, y, jnp.exp(jnp.minimum(y, 0.0)) - 1.0)
    o_ref[...] = y


def _mm(x, w, b, act=None, block=2000):
    m, k = x.shape
    n = w.shape[1]
    assert m % block == 0
    return pl.pallas_call(
        functools.partial(_mm_body, act=act),
        grid=(m // block,),
        in_specs=[
            pl.BlockSpec((block, k), lambda i: (i, 0)),
            pl.BlockSpec((k, n), lambda i: (0, 0)),
            pl.BlockSpec((n,), lambda i: (0,)),
        ],
        out_specs=pl.BlockSpec((block, n), lambda i: (i, 0)),
        out_shape=jax.ShapeDtypeStruct((m, n), jnp.float32),
    )(x, w, b)


def _gat_edge(p, xl, xr, ei, ea, n_dst):
    src, dst = ei[0], ei[1]
    ee = (ea @ p["we"]).reshape(-1, HEADS, HD)
    xl3 = xl.reshape(-1, HEADS, HD)
    xr3 = xr.reshape(-1, HEADS, HD)
    h = xl3[src] + xr3[dst] + ee
    h = jax.nn.leaky_relu(h, 0.2)
    alpha = jnp.sum(h * p["att"][None, :, :], axis=-1)
    amax = jax.ops.segment_max(alpha, dst, num_segments=n_dst)
    amax = jnp.where(jnp.isfinite(amax), amax, 0.0)
    ex = jnp.exp(alpha - amax[dst])
    den = jax.ops.segment_sum(ex, dst, num_segments=n_dst)
    a = ex / (den[dst] + 1e-16)
    msg = xl3[src] * a[:, :, None]
    out = jax.ops.segment_sum(msg, dst, num_segments=n_dst).reshape(n_dst, HEADS * HD)
    return out + p["bias"]


def _gatv2(p, x_src, x_dst, ei, ea, n_dst):
    xl = _mm(x_src, p["wl"], p["bl"])
    xr = _mm(x_dst, p["wr"], p["br"])
    return _gat_edge(p, xl, xr, ei, ea, n_dst)


def _bn_elu(p, x):
    y = jax.nn.elu(x)
    m = jnp.mean(y, axis=0)
    v = jnp.var(y, axis=0)
    return (y - m) / jnp.sqrt(v + 1e-5) * p["g"] + p["b"]


def kernel(x_lidar, x_radar1, x_radar2, ei_ll, ei_r1r1, ei_r2r2, ei_lr1, ei_lr2,
           ea_ll, ea_r1r1, ea_r2r2, ea_lr1, ea_lr2, params):
    x = {
        "lidar": _mm(x_lidar, params["proj"]["lidar"]["w"], params["proj"]["lidar"]["b"], act="elu"),
        "radar1": _mm(x_radar1, params["proj"]["radar1"]["w"], params["proj"]["radar1"]["b"], act="elu"),
        "radar2": _mm(x_radar2, params["proj"]["radar2"]["w"], params["proj"]["radar2"]["b"], act="elu"),
    }
    for layer in params["layers"]:
        c = layer["conv"]
        o_l = _gatv2(c["ll"], x["lidar"], x["lidar"], ei_ll, ea_ll, N_L)
        o_r1 = (_gatv2(c["r1r1"], x["radar1"], x["radar1"], ei_r1r1, ea_r1r1, N_L)
                + _gatv2(c["lr1"], x["lidar"], x["radar1"], ei_lr1, ea_lr1, N_L))
        o_r2 = (_gatv2(c["r2r2"], x["radar2"], x["radar2"], ei_r2r2, ea_r2r2, N_L)
                + _gatv2(c["lr2"], x["lidar"], x["radar2"], ei_lr2, ea_lr2, N_L))
        x = {
            "lidar": _bn_elu(layer["bn"]["lidar"], o_l),
            "radar1": _bn_elu(layer["bn"]["radar1"], o_r1),
            "radar2": _bn_elu(layer["bn"]["radar2"], o_r2),
        }
    out_l = x["lidar"] @ params["head_lidar"]["w"] + params["head_lidar"]["b"]
    out_r1 = x["radar1"] @ params["head_radar"]["w"] + params["head_radar"]["b"]
    out_r2 = x["radar2"] @ params["head_radar"]["w"] + params["head_radar"]["b"]
    return (out_l, out_r1, out_r2)


# speedup vs baseline: 18.6590x; 18.6373x over previous
import functools
import jax
import jax.numpy as jnp
from jax import lax
from jax.experimental import pallas as pl
from jax.experimental.pallas import tpu as pltpu
from jax.experimental.pallas import tpu_sc as plsc

N = 10000
E = 160000
HID = 256
HALF = 128
B = 80
EPT = E // 16
NBLK = EPT // B
NEG = -1e30


def _sc_body(xl_hbm, xr_hbm, ee_hbm, src_hbm, dst_hbm, att_hbm, out_hbm, al_hbm,
             isrc, idst, igs, ixr, idx0, idx1, idx2, idx3,
             exb0, exb1, exb2, exb3, den0, den1, den2, den3,
             xlb, xrb, eeb, alb2, zb, attb, mxst, mxrd,
             acc, den_sp, maxstage, sem0, sem1, sem2):
    idx4 = [idx0, idx1, idx2, idx3]
    exb4 = [exb0, exb1, exb2, exb3]
    denb4 = [den0, den1, den2, den3]
    c = lax.axis_index("c")
    s = lax.axis_index("s")
    cbase = c * N
    ebase = s * EPT
    zv = jnp.zeros((16,), jnp.float32)

    def zrow(i, _):
        for m in range(8):
            xlb[i, pl.ds(16 * m, 16)] = zv
        return 0
    lax.fori_loop(0, B, zrow, 0)
    def zzb(i, _):
        zb[pl.ds(i * 16, 16)] = zv
        return 0
    lax.fori_loop(0, 160, zzb, 0)
    pltpu.sync_copy(att_hbm.at[pl.ds(c * HALF, HALF)], attb)

    nchunk = jnp.where(s < 15, 8, 5)
    def zchunk(k, _):
        pltpu.sync_copy(xlb, acc.at[pl.ds(pl.multiple_of(s * 640 + k * 80, 8), 80)])
        return 0
    lax.fori_loop(0, nchunk, zchunk, 0)
    pltpu.sync_copy(zb, den_sp.at[pl.ds(pl.multiple_of(s * 2560, 128), 2560)])
    plsc.subcore_barrier()

    lanes = lax.broadcasted_iota(jnp.int32, (16,), 0)

    def p1_blk(j, mv):
        base = ebase + j * B
        pltpu.sync_copy(src_hbm.at[pl.ds(base, B)], isrc)
        pltpu.sync_copy(dst_hbm.at[pl.ds(base, B)], idst)
        for t in range(5):
            igs[pl.ds(16 * t, 16)] = isrc[pl.ds(16 * t, 16)] + cbase
            ixr[pl.ds(16 * t, 16)] = idst[pl.ds(16 * t, 16)] + cbase
        cpa = pltpu.async_copy(xl_hbm.at[igs], xlb, sem0)
        cpb = pltpu.async_copy(xr_hbm.at[ixr], xrb, sem1)
        cpc = pltpu.async_copy(ee_hbm.at[pl.ds(c * E + base, B)], eeb, sem2)
        cpa.wait()
        cpb.wait()
        cpc.wait()

        for t in range(5):
            def pedge(i, carry):
                e = 16 * t + i
                outs = []
                for k in range(4):
                    z0 = xlb[e, pl.ds(32 * k, 16)] + xrb[e, pl.ds(32 * k, 16)] + eeb[e, pl.ds(32 * k, 16)]
                    z1 = xlb[e, pl.ds(32 * k + 16, 16)] + xrb[e, pl.ds(32 * k + 16, 16)] + eeb[e, pl.ds(32 * k + 16, 16)]
                    h0 = jnp.where(z0 > 0, z0, 0.2 * z0)
                    h1 = jnp.where(z1 > 0, z1, 0.2 * z1)
                    ts = h0 * attb[pl.ds(32 * k, 16)] + h1 * attb[pl.ds(32 * k + 16, 16)]
                    for st in (1, 2, 4, 8):
                        ts = ts + ts.at[lanes ^ st].get(mode="promise_in_bounds")
                    outs.append(jnp.where(lanes == i, ts, carry[k]))
                return tuple(outs)
            a = lax.fori_loop(0, 16, pedge, (zv, zv, zv, zv))
            for k in range(4):
                alb2[k, pl.ds(16 * t, 16)] = a[k]
                mv = jnp.maximum(mv, a[k])
        pltpu.sync_copy(alb2, al_hbm.at[c].at[s].at[j])
        return mv
    mv = lax.fori_loop(0, NBLK, p1_blk, jnp.full((16,), NEG, jnp.float32))

    for m in range(8):
        mxst[pl.ds(16 * m, 16)] = mv
    pltpu.sync_copy(mxst, maxstage.at[s])
    plsc.subcore_barrier()
    pltpu.sync_copy(maxstage, mxrd)
    gv = mxrd[0, pl.ds(0, 16)]
    for t in range(1, 16):
        gv = jnp.maximum(gv, mxrd[t, pl.ds(0, 16)])
    for st in (1, 2, 4, 8):
        gv = jnp.maximum(gv, gv.at[lanes ^ st].get(mode="promise_in_bounds"))
    G = gv

    def p2_blk(j, _):
        base = ebase + j * B
        pltpu.sync_copy(dst_hbm.at[pl.ds(base, B)], idst)
        pltpu.sync_copy(al_hbm.at[c].at[s].at[j], alb2)
        for k in range(4):
            for t in range(5):
                idx4[k][pl.ds(16 * t, 16)] = idst[pl.ds(16 * t, 16)] + (N * k)
                exb4[k][pl.ds(16 * t, 16)] = jnp.exp(alb2[k, pl.ds(16 * t, 16)] - G)
            pltpu.sync_copy(exb4[k], den_sp.at[idx4[k]], add=True)
        return 0
    lax.fori_loop(0, NBLK, p2_blk, 0)
    plsc.subcore_barrier()

    def p3_blk(j, _):
        base = ebase + j * B
        pltpu.sync_copy(src_hbm.at[pl.ds(base, B)], isrc)
        pltpu.sync_copy(dst_hbm.at[pl.ds(base, B)], idst)
        for t in range(5):
            igs[pl.ds(16 * t, 16)] = isrc[pl.ds(16 * t, 16)] + cbase
        cpa = pltpu.async_copy(xl_hbm.at[igs], xlb, sem0)
        pltpu.sync_copy(al_hbm.at[c].at[s].at[j], alb2)
        for k in range(4):
            for t in range(5):
                idx4[k][pl.ds(16 * t, 16)] = idst[pl.ds(16 * t, 16)] + (N * k)
            pltpu.sync_copy(den_sp.at[idx4[k]], denb4[k])
        for k in range(4):
            for t in range(5):
                denb4[k][pl.ds(16 * t, 16)] = (
                    jnp.exp(alb2[k, pl.ds(16 * t, 16)] - G)
                    / (denb4[k][pl.ds(16 * t, 16)] + 1e-16))
        cpa.wait()

        for t in range(5):
            a16 = [denb4[k][pl.ds(16 * t, 16)] for k in range(4)]

            def pedge(i, _):
                e = 16 * t + i
                sel = lanes * 0 + i
                for k in range(4):
                    av = a16[k].at[sel].get(mode="promise_in_bounds")
                    xlb[e, pl.ds(32 * k, 16)] = xlb[e, pl.ds(32 * k, 16)] * av
                    xlb[e, pl.ds(32 * k + 16, 16)] = xlb[e, pl.ds(32 * k + 16, 16)] * av
                return 0
            lax.fori_loop(0, 16, pedge, 0)
        pltpu.sync_copy(xlb, acc.at[idst], add=True)
        return 0
    lax.fori_loop(0, NBLK, p3_blk, 0)
    plsc.subcore_barrier()

    @pl.when(s < 15)
    def _():
        pltpu.sync_copy(acc.at[pl.ds(pl.multiple_of(s * 640, 8), 640)],
                        out_hbm.at[pl.ds(pl.multiple_of(cbase + s * 640, 8), 640)])
    @pl.when(s == 15)
    def _():
        pltpu.sync_copy(acc.at[pl.ds(9600, 400)],
                        out_hbm.at[pl.ds(pl.multiple_of(cbase + 9600, 8), 400)])


@jax.jit
def _sc_edge(xl_cat, xr_cat, ee_cat, src, dst, att_flat):
    mesh = plsc.VectorSubcoreMesh(core_axis_name="c", subcore_axis_name="s")
    f = pl.kernel(
        _sc_body,
        out_type=[
            jax.ShapeDtypeStruct((2 * N, HALF), jnp.float32),
            jax.ShapeDtypeStruct((2, 16, NBLK, 4, B), jnp.float32),
        ],
        mesh=mesh,
        scratch_types=(
            [pltpu.VMEM((B,), jnp.int32)] * 4
            + [pltpu.VMEM((B,), jnp.int32)] * 4
            + [pltpu.VMEM((B,), jnp.float32)] * 8
            + [
                pltpu.VMEM((B, HALF), jnp.float32),
                pltpu.VMEM((B, HALF), jnp.float32),
                pltpu.VMEM((B, HALF), jnp.float32),
                pltpu.VMEM((4, B), jnp.float32),
                pltpu.VMEM((2560,), jnp.float32),
                pltpu.VMEM((HALF,), jnp.float32),
                pltpu.VMEM((HALF,), jnp.float32),
                pltpu.VMEM((16, HALF), jnp.float32),
                pltpu.VMEM_SHARED((N, HALF), jnp.float32),
                pltpu.VMEM_SHARED((40960,), jnp.float32),
                pltpu.VMEM_SHARED((16, HALF), jnp.float32),
                pltpu.SemaphoreType.DMA,
                pltpu.SemaphoreType.DMA,
                pltpu.SemaphoreType.DMA,
            ]
        ),
    )
    return f(xl_cat, xr_cat, ee_cat, src, dst, att_flat)[0]


def _proj_body(x_ref, w_ref, b_ref, o_ref):
    y = jnp.dot(x_ref[...], w_ref[0], preferred_element_type=jnp.float32)
    y = y + b_ref[0, 0, :]
    o_ref[0] = jnp.where(y > 0, y, jnp.exp(jnp.minimum(y, 0.0)) - 1.0)


def _proj(x, w, b):
    m, k = x.shape
    blk = 2000
    return pl.pallas_call(
        _proj_body,
        grid=(m // blk, 2),
        in_specs=[
            pl.BlockSpec((blk, k), lambda i, h: (i, 0)),
            pl.BlockSpec((1, k, HALF), lambda i, h: (h, 0, 0)),
            pl.BlockSpec((1, 1, HALF), lambda i, h: (h, 0, 0)),
        ],
        out_specs=pl.BlockSpec((1, blk, HALF), lambda i, h: (h, i, 0)),
        out_shape=jax.ShapeDtypeStruct((2, m, HALF), jnp.float32),
    )(x, jnp.transpose(w.reshape(k, 2, HALF), (1, 0, 2)), b.reshape(2, 1, HALF))


def _lin_body(x_ref, w_ref, b_ref, o_ref):
    y = jnp.dot(x_ref[0], w_ref[0, 0], preferred_element_type=jnp.float32)
    y = y + jnp.dot(x_ref[1], w_ref[0, 1], preferred_element_type=jnp.float32)
    o_ref[0] = y + b_ref[0, 0, :]


def _lin_split(x2, w, b):
    m = x2.shape[1]
    blk = 2000
    return pl.pallas_call(
        _lin_body,
        grid=(m // blk, 2),
        in_specs=[
            pl.BlockSpec((2, blk, HALF), lambda i, h: (0, i, 0)),
            pl.BlockSpec((1, 2, HALF, HALF), lambda i, h: (h, 0, 0, 0)),
            pl.BlockSpec((1, 1, HALF), lambda i, h: (h, 0, 0)),
        ],
        out_specs=pl.BlockSpec((1, blk, HALF), lambda i, h: (h, i, 0)),
        out_shape=jax.ShapeDtypeStruct((2, m, HALF), jnp.float32),
    )(x2, jnp.transpose(w.reshape(2, HALF, 2, HALF), (2, 0, 1, 3)),
      b.reshape(2, 1, HALF))


def _ee_body(ea_ref, we_ref, o_ref):
    ea = ea_ref[...]
    y = ea[:, 0][:, None] * we_ref[0, 0, :][None, :]
    for j in range(1, 4):
        y = y + ea[:, j][:, None] * we_ref[0, j, :][None, :]
    o_ref[0] = y


def _ee(ea, we):
    m = ea.shape[0]
    blk = 2000
    return pl.pallas_call(
        _ee_body,
        grid=(m // blk, 2),
        in_specs=[
            pl.BlockSpec((blk, 4), lambda i, h: (i, 0)),
            pl.BlockSpec((1, 4, HALF), lambda i, h: (h, 0, 0)),
        ],
        out_specs=pl.BlockSpec((1, blk, HALF), lambda i, h: (h, i, 0)),
        out_shape=jax.ShapeDtypeStruct((2, m, HALF), jnp.float32),
    )(ea, jnp.transpose(we.reshape(4, 2, HALF), (1, 0, 2)))


def _bn1_body(o1_ref, bias_ref, g_ref, b_ref, o_ref):
    x = o1_ref[0] + bias_ref[0, 0, :]
    y = jnp.where(x > 0, x, jnp.exp(jnp.minimum(x, 0.0)) - 1.0)
    m = jnp.mean(y, axis=0)
    v = jnp.mean((y - m[None, :]) ** 2, axis=0)
    o_ref[0] = (y - m[None, :]) * jax.lax.rsqrt(v + 1e-5)[None, :] * g_ref[0, 0, :] + b_ref[0, 0, :]


def _bn2_body(o1_ref, o2_ref, bias_ref, g_ref, b_ref, o_ref):
    x = o1_ref[0] + o2_ref[0] + bias_ref[0, 0, :]
    y = jnp.where(x > 0, x, jnp.exp(jnp.minimum(x, 0.0)) - 1.0)
    m = jnp.mean(y, axis=0)
    v = jnp.mean((y - m[None, :]) ** 2, axis=0)
    o_ref[0] = (y - m[None, :]) * jax.lax.rsqrt(v + 1e-5)[None, :] * g_ref[0, 0, :] + b_ref[0, 0, :]


def _bn(o1, o2, bias, g, b):
    vec = pl.BlockSpec((1, 1, HALF), lambda h: (h, 0, 0))
    full = pl.BlockSpec((1, N, HALF), lambda h: (h, 0, 0))
    if o2 is None:
        return pl.pallas_call(
            _bn1_body, grid=(2,),
            in_specs=[full, vec, vec, vec],
            out_specs=full,
            out_shape=jax.ShapeDtypeStruct((2, N, HALF), jnp.float32),
        )(o1, bias.reshape(2, 1, HALF), g.reshape(2, 1, HALF), b.reshape(2, 1, HALF))
    return pl.pallas_call(
        _bn2_body, grid=(2,),
        in_specs=[full, full, vec, vec, vec],
        out_specs=full,
        out_shape=jax.ShapeDtypeStruct((2, N, HALF), jnp.float32),
    )(o1, o2, bias.reshape(2, 1, HALF), g.reshape(2, 1, HALF), b.reshape(2, 1, HALF))


def _head_body(x_ref, w_ref, b_ref, o_ref):
    y = jnp.dot(x_ref[0], w_ref[0][:, None], preferred_element_type=jnp.float32)
    y = y + jnp.dot(x_ref[1], w_ref[1][:, None], preferred_element_type=jnp.float32)
    o_ref[...] = y + b_ref[0]


def _head(x2, w, b):
    blk = 2000
    return pl.pallas_call(
        _head_body,
        grid=(N // blk,),
        in_specs=[
            pl.BlockSpec((2, blk, HALF), lambda i: (0, i, 0)),
            pl.BlockSpec((2, HALF), lambda i: (0, 0)),
            pl.BlockSpec((1,), lambda i: (0,)),
        ],
        out_specs=pl.BlockSpec((blk, 1), lambda i: (i, 0)),
        out_shape=jax.ShapeDtypeStruct((N, 1), jnp.float32),
    )(x2, w.reshape(2, HALF), b)


def _gatv2_sc(p, x_src2, x_dst2, ei, ea):
    xl = _lin_split(x_src2, p["wl"], p["bl"])
    xr = _lin_split(x_dst2, p["wr"], p["br"])
    ee = _ee(ea, p["we"])
    o = _sc_edge(xl.reshape(2 * N, HALF), xr.reshape(2 * N, HALF),
                 ee.reshape(2 * E, HALF), ei[0], ei[1],
                 p["att"].reshape(HID))
    return o.reshape(2, N, HALF)


def kernel(x_lidar, x_radar1, x_radar2, ei_ll, ei_r1r1, ei_r2r2, ei_lr1, ei_lr2,
           ea_ll, ea_r1r1, ea_r2r2, ea_lr1, ea_lr2, params):
    x = {
        "lidar": _proj(x_lidar, params["proj"]["lidar"]["w"], params["proj"]["lidar"]["b"]),
        "radar1": _proj(x_radar1, params["proj"]["radar1"]["w"], params["proj"]["radar1"]["b"]),
        "radar2": _proj(x_radar2, params["proj"]["radar2"]["w"], params["proj"]["radar2"]["b"]),
    }
    for layer in params["layers"]:
        c = layer["conv"]
        o_ll = _gatv2_sc(c["ll"], x["lidar"], x["lidar"], ei_ll, ea_ll)
        o_r1r1 = _gatv2_sc(c["r1r1"], x["radar1"], x["radar1"], ei_r1r1, ea_r1r1)
        o_lr1 = _gatv2_sc(c["lr1"], x["lidar"], x["radar1"], ei_lr1, ea_lr1)
        o_r2r2 = _gatv2_sc(c["r2r2"], x["radar2"], x["radar2"], ei_r2r2, ea_r2r2)
        o_lr2 = _gatv2_sc(c["lr2"], x["lidar"], x["radar2"], ei_lr2, ea_lr2)
        bn = layer["bn"]
        x = {
            "lidar": _bn(o_ll, None, c["ll"]["bias"], bn["lidar"]["g"], bn["lidar"]["b"]),
            "radar1": _bn(o_r1r1, o_lr1, c["r1r1"]["bias"] + c["lr1"]["bias"],
                          bn["radar1"]["g"], bn["radar1"]["b"]),
            "radar2": _bn(o_r2r2, o_lr2, c["r2r2"]["bias"] + c["lr2"]["bias"],
                          bn["radar2"]["g"], bn["radar2"]["b"]),
        }
    out_l = _head(x["lidar"], params["head_lidar"]["w"], params["head_lidar"]["b"])
    out_r1 = _head(x["radar1"], params["head_radar"]["w"], params["head_radar"]["b"])
    out_r2 = _head(x["radar2"], params["head_radar"]["w"], params["head_radar"]["b"])
    return (out_l, out_r1, out_r2)


# speedup vs baseline: 18.7991x; 1.0075x over previous
import functools
import jax
import jax.numpy as jnp
from jax import lax
from jax.experimental import pallas as pl
from jax.experimental.pallas import tpu as pltpu
from jax.experimental.pallas import tpu_sc as plsc

N = 10000
E = 160000
HID = 256
HALF = 128
B = 80
EPT = E // 16
NBLK = EPT // B
NEG = -1e30


def _sc_body(xl_hbm, xr_hbm, ee_hbm, src_hbm, dst_hbm, att_hbm, out_hbm, al_hbm,
             isrc, idst, igs, ixr, idx0, idx1, idx2, idx3,
             exb0, exb1, exb2, exb3, den0, den1, den2, den3,
             xlb, xrb, eeb, alb2, zb, attb, mxst, mxrd,
             acc, den_sp, maxstage, sem0, sem1, sem2):
    idx4 = [idx0, idx1, idx2, idx3]
    exb4 = [exb0, exb1, exb2, exb3]
    denb4 = [den0, den1, den2, den3]
    c = lax.axis_index("c")
    s = lax.axis_index("s")
    cbase = c * N
    ebase = s * EPT
    zv = jnp.zeros((16,), jnp.float32)

    def zrow(i, _):
        for m in range(8):
            xlb[i, pl.ds(16 * m, 16)] = zv
        return 0
    lax.fori_loop(0, B, zrow, 0)
    def zzb(i, _):
        zb[pl.ds(i * 16, 16)] = zv
        return 0
    lax.fori_loop(0, 160, zzb, 0)
    pltpu.sync_copy(att_hbm.at[pl.ds(c * HALF, HALF)], attb)

    nchunk = jnp.where(s < 15, 8, 5)
    def zchunk(k, _):
        pltpu.sync_copy(xlb, acc.at[pl.ds(pl.multiple_of(s * 640 + k * 80, 8), 80)])
        return 0
    lax.fori_loop(0, nchunk, zchunk, 0)
    pltpu.sync_copy(zb, den_sp.at[pl.ds(pl.multiple_of(s * 2560, 128), 2560)])
    plsc.subcore_barrier()

    lanes = lax.broadcasted_iota(jnp.int32, (16,), 0)

    def p1_blk(j, mv):
        base = ebase + j * B
        pltpu.sync_copy(src_hbm.at[pl.ds(base, B)], isrc)
        pltpu.sync_copy(dst_hbm.at[pl.ds(base, B)], idst)
        for t in range(5):
            igs[pl.ds(16 * t, 16)] = isrc[pl.ds(16 * t, 16)] + cbase
            ixr[pl.ds(16 * t, 16)] = idst[pl.ds(16 * t, 16)] + cbase
        cpa = pltpu.async_copy(xl_hbm.at[igs], xlb, sem0)
        cpb = pltpu.async_copy(xr_hbm.at[ixr], xrb, sem1)
        cpc = pltpu.async_copy(ee_hbm.at[pl.ds(c * E + base, B)], eeb, sem2)
        cpa.wait()
        cpb.wait()
        cpc.wait()

        for t in range(5):
            def pedge(i, carry):
                e = 16 * t + i
                outs = []
                for k in range(4):
                    z0 = xlb[e, pl.ds(32 * k, 16)] + xrb[e, pl.ds(32 * k, 16)] + eeb[e, pl.ds(32 * k, 16)]
                    z1 = xlb[e, pl.ds(32 * k + 16, 16)] + xrb[e, pl.ds(32 * k + 16, 16)] + eeb[e, pl.ds(32 * k + 16, 16)]
                    h0 = jnp.maximum(z0, 0.2 * z0)
                    h1 = jnp.maximum(z1, 0.2 * z1)
                    ts = h0 * attb[pl.ds(32 * k, 16)] + h1 * attb[pl.ds(32 * k + 16, 16)]
                    for st in (1, 2, 4, 8):
                        ts = ts + ts.at[lanes ^ st].get(mode="promise_in_bounds")
                    outs.append(jnp.where(lanes == i, ts, carry[k]))
                return tuple(outs)
            a = lax.fori_loop(0, 16, pedge, (zv, zv, zv, zv))
            for k in range(4):
                alb2[k, pl.ds(16 * t, 16)] = a[k]
                mv = jnp.maximum(mv, a[k])
        pltpu.sync_copy(alb2, al_hbm.at[c].at[s].at[j])
        return mv
    mv = lax.fori_loop(0, NBLK, p1_blk, jnp.full((16,), NEG, jnp.float32))

    for m in range(8):
        mxst[pl.ds(16 * m, 16)] = mv
    pltpu.sync_copy(mxst, maxstage.at[s])
    plsc.subcore_barrier()
    pltpu.sync_copy(maxstage, mxrd)
    gv = mxrd[0, pl.ds(0, 16)]
    for t in range(1, 16):
        gv = jnp.maximum(gv, mxrd[t, pl.ds(0, 16)])
    for st in (1, 2, 4, 8):
        gv = jnp.maximum(gv, gv.at[lanes ^ st].get(mode="promise_in_bounds"))
    G = gv

    def p2_blk(j, _):
        base = ebase + j * B
        pltpu.sync_copy(dst_hbm.at[pl.ds(base, B)], idst)
        pltpu.sync_copy(al_hbm.at[c].at[s].at[j], alb2)
        for k in range(4):
            for t in range(5):
                idx4[k][pl.ds(16 * t, 16)] = idst[pl.ds(16 * t, 16)] + (N * k)
                exb4[k][pl.ds(16 * t, 16)] = jnp.exp(alb2[k, pl.ds(16 * t, 16)] - G)
            pltpu.sync_copy(exb4[k], den_sp.at[idx4[k]], add=True)
        return 0
    lax.fori_loop(0, NBLK, p2_blk, 0)
    plsc.subcore_barrier()

    def p3_blk(j, _):
        base = ebase + j * B
        pltpu.sync_copy(src_hbm.at[pl.ds(base, B)], isrc)
        pltpu.sync_copy(dst_hbm.at[pl.ds(base, B)], idst)
        for t in range(5):
            igs[pl.ds(16 * t, 16)] = isrc[pl.ds(16 * t, 16)] + cbase
        cpa = pltpu.async_copy(xl_hbm.at[igs], xlb, sem0)
        pltpu.sync_copy(al_hbm.at[c].at[s].at[j], alb2)
        for k in range(4):
            for t in range(5):
                idx4[k][pl.ds(16 * t, 16)] = idst[pl.ds(16 * t, 16)] + (N * k)
            pltpu.sync_copy(den_sp.at[idx4[k]], denb4[k])
        for k in range(4):
            for t in range(5):
                denb4[k][pl.ds(16 * t, 16)] = (
                    jnp.exp(alb2[k, pl.ds(16 * t, 16)] - G)
                    / (denb4[k][pl.ds(16 * t, 16)] + 1e-16))
        cpa.wait()

        for t in range(5):
            a16 = [denb4[k][pl.ds(16 * t, 16)] for k in range(4)]

            def pedge(i, _):
                e = 16 * t + i
                sel = lanes * 0 + i
                for k in range(4):
                    av = a16[k].at[sel].get(mode="promise_in_bounds")
                    xlb[e, pl.ds(32 * k, 16)] = xlb[e, pl.ds(32 * k, 16)] * av
                    xlb[e, pl.ds(32 * k + 16, 16)] = xlb[e, pl.ds(32 * k + 16, 16)] * av
                return 0
            lax.fori_loop(0, 16, pedge, 0)
        pltpu.sync_copy(xlb, acc.at[idst], add=True)
        return 0
    lax.fori_loop(0, NBLK, p3_blk, 0)
    plsc.subcore_barrier()

    @pl.when(s < 15)
    def _():
        pltpu.sync_copy(acc.at[pl.ds(pl.multiple_of(s * 640, 8), 640)],
                        out_hbm.at[pl.ds(pl.multiple_of(cbase + s * 640, 8), 640)])
    @pl.when(s == 15)
    def _():
        pltpu.sync_copy(acc.at[pl.ds(9600, 400)],
                        out_hbm.at[pl.ds(pl.multiple_of(cbase + 9600, 8), 400)])


@jax.jit
def _sc_edge(xl_cat, xr_cat, ee_cat, src, dst, att_flat):
    mesh = plsc.VectorSubcoreMesh(core_axis_name="c", subcore_axis_name="s")
    f = pl.kernel(
        _sc_body,
        out_type=[
            jax.ShapeDtypeStruct((2 * N, HALF), jnp.float32),
            jax.ShapeDtypeStruct((2, 16, NBLK, 4, B), jnp.float32),
        ],
        mesh=mesh,
        scratch_types=(
            [pltpu.VMEM((B,), jnp.int32)] * 4
            + [pltpu.VMEM((B,), jnp.int32)] * 4
            + [pltpu.VMEM((B,), jnp.float32)] * 8
            + [
                pltpu.VMEM((B, HALF), jnp.float32),
                pltpu.VMEM((B, HALF), jnp.float32),
                pltpu.VMEM((B, HALF), jnp.float32),
                pltpu.VMEM((4, B), jnp.float32),
                pltpu.VMEM((2560,), jnp.float32),
                pltpu.VMEM((HALF,), jnp.float32),
                pltpu.VMEM((HALF,), jnp.float32),
                pltpu.VMEM((16, HALF), jnp.float32),
                pltpu.VMEM_SHARED((N, HALF), jnp.float32),
                pltpu.VMEM_SHARED((40960,), jnp.float32),
                pltpu.VMEM_SHARED((16, HALF), jnp.float32),
                pltpu.SemaphoreType.DMA,
                pltpu.SemaphoreType.DMA,
                pltpu.SemaphoreType.DMA,
            ]
        ),
    )
    return f(xl_cat, xr_cat, ee_cat, src, dst, att_flat)[0]


def _proj_body(x_ref, w_ref, b_ref, o_ref):
    y = jnp.dot(x_ref[...], w_ref[0], preferred_element_type=jnp.float32)
    y = y + b_ref[0, 0, :]
    o_ref[0] = jnp.where(y > 0, y, jnp.exp(jnp.minimum(y, 0.0)) - 1.0)


def _proj(x, w, b):
    m, k = x.shape
    blk = 2000
    return pl.pallas_call(
        _proj_body,
        grid=(m // blk, 2),
        in_specs=[
            pl.BlockSpec((blk, k), lambda i, h: (i, 0)),
            pl.BlockSpec((1, k, HALF), lambda i, h: (h, 0, 0)),
            pl.BlockSpec((1, 1, HALF), lambda i, h: (h, 0, 0)),
        ],
        out_specs=pl.BlockSpec((1, blk, HALF), lambda i, h: (h, i, 0)),
        out_shape=jax.ShapeDtypeStruct((2, m, HALF), jnp.float32),
    )(x, jnp.transpose(w.reshape(k, 2, HALF), (1, 0, 2)), b.reshape(2, 1, HALF))


def _lin_body(x_ref, w_ref, b_ref, o_ref):
    y = jnp.dot(x_ref[0], w_ref[0, 0], preferred_element_type=jnp.float32)
    y = y + jnp.dot(x_ref[1], w_ref[0, 1], preferred_element_type=jnp.float32)
    o_ref[0] = y + b_ref[0, 0, :]


def _lin_split(x2, w, b):
    m = x2.shape[1]
    blk = 2000
    return pl.pallas_call(
        _lin_body,
        grid=(m // blk, 2),
        in_specs=[
            pl.BlockSpec((2, blk, HALF), lambda i, h: (0, i, 0)),
            pl.BlockSpec((1, 2, HALF, HALF), lambda i, h: (h, 0, 0, 0)),
            pl.BlockSpec((1, 1, HALF), lambda i, h: (h, 0, 0)),
        ],
        out_specs=pl.BlockSpec((1, blk, HALF), lambda i, h: (h, i, 0)),
        out_shape=jax.ShapeDtypeStruct((2, m, HALF), jnp.float32),
    )(x2, jnp.transpose(w.reshape(2, HALF, 2, HALF), (2, 0, 1, 3)),
      b.reshape(2, 1, HALF))


def _ee_body(ea_ref, we_ref, o_ref):
    ea = ea_ref[...]
    y = ea[:, 0][:, None] * we_ref[0, 0, :][None, :]
    for j in range(1, 4):
        y = y + ea[:, j][:, None] * we_ref[0, j, :][None, :]
    o_ref[0] = y


def _ee(ea, we):
    m = ea.shape[0]
    blk = 2000
    return pl.pallas_call(
        _ee_body,
        grid=(m // blk, 2),
        in_specs=[
            pl.BlockSpec((blk, 4), lambda i, h: (i, 0)),
            pl.BlockSpec((1, 4, HALF), lambda i, h: (h, 0, 0)),
        ],
        out_specs=pl.BlockSpec((1, blk, HALF), lambda i, h: (h, i, 0)),
        out_shape=jax.ShapeDtypeStruct((2, m, HALF), jnp.float32),
    )(ea, jnp.transpose(we.reshape(4, 2, HALF), (1, 0, 2)))


def _bn1_body(o1_ref, bias_ref, g_ref, b_ref, o_ref):
    x = o1_ref[0] + bias_ref[0, 0, :]
    y = jnp.where(x > 0, x, jnp.exp(jnp.minimum(x, 0.0)) - 1.0)
    m = jnp.mean(y, axis=0)
    v = jnp.mean((y - m[None, :]) ** 2, axis=0)
    o_ref[0] = (y - m[None, :]) * jax.lax.rsqrt(v + 1e-5)[None, :] * g_ref[0, 0, :] + b_ref[0, 0, :]


def _bn2_body(o1_ref, o2_ref, bias_ref, g_ref, b_ref, o_ref):
    x = o1_ref[0] + o2_ref[0] + bias_ref[0, 0, :]
    y = jnp.where(x > 0, x, jnp.exp(jnp.minimum(x, 0.0)) - 1.0)
    m = jnp.mean(y, axis=0)
    v = jnp.mean((y - m[None, :]) ** 2, axis=0)
    o_ref[0] = (y - m[None, :]) * jax.lax.rsqrt(v + 1e-5)[None, :] * g_ref[0, 0, :] + b_ref[0, 0, :]


def _bn(o1, o2, bias, g, b):
    vec = pl.BlockSpec((1, 1, HALF), lambda h: (h, 0, 0))
    full = pl.BlockSpec((1, N, HALF), lambda h: (h, 0, 0))
    if o2 is None:
        return pl.pallas_call(
            _bn1_body, grid=(2,),
            in_specs=[full, vec, vec, vec],
            out_specs=full,
            out_shape=jax.ShapeDtypeStruct((2, N, HALF), jnp.float32),
        )(o1, bias.reshape(2, 1, HALF), g.reshape(2, 1, HALF), b.reshape(2, 1, HALF))
    return pl.pallas_call(
        _bn2_body, grid=(2,),
        in_specs=[full, full, vec, vec, vec],
        out_specs=full,
        out_shape=jax.ShapeDtypeStruct((2, N, HALF), jnp.float32),
    )(o1, o2, bias.reshape(2, 1, HALF), g.reshape(2, 1, HALF), b.reshape(2, 1, HALF))


def _head_body(x_ref, w_ref, b_ref, o_ref):
    y = jnp.dot(x_ref[0], w_ref[0][:, None], preferred_element_type=jnp.float32)
    y = y + jnp.dot(x_ref[1], w_ref[1][:, None], preferred_element_type=jnp.float32)
    o_ref[...] = y + b_ref[0]


def _head(x2, w, b):
    blk = 2000
    return pl.pallas_call(
        _head_body,
        grid=(N // blk,),
        in_specs=[
            pl.BlockSpec((2, blk, HALF), lambda i: (0, i, 0)),
            pl.BlockSpec((2, HALF), lambda i: (0, 0)),
            pl.BlockSpec((1,), lambda i: (0,)),
        ],
        out_specs=pl.BlockSpec((blk, 1), lambda i: (i, 0)),
        out_shape=jax.ShapeDtypeStruct((N, 1), jnp.float32),
    )(x2, w.reshape(2, HALF), b)


def _gatv2_sc(p, x_src2, x_dst2, ei, ea):
    xl = _lin_split(x_src2, p["wl"], p["bl"])
    xr = _lin_split(x_dst2, p["wr"], p["br"])
    ee = _ee(ea, p["we"])
    o = _sc_edge(xl.reshape(2 * N, HALF), xr.reshape(2 * N, HALF),
                 ee.reshape(2 * E, HALF), ei[0], ei[1],
                 p["att"].reshape(HID))
    return o.reshape(2, N, HALF)


def kernel(x_lidar, x_radar1, x_radar2, ei_ll, ei_r1r1, ei_r2r2, ei_lr1, ei_lr2,
           ea_ll, ea_r1r1, ea_r2r2, ea_lr1, ea_lr2, params):
    x = {
        "lidar": _proj(x_lidar, params["proj"]["lidar"]["w"], params["proj"]["lidar"]["b"]),
        "radar1": _proj(x_radar1, params["proj"]["radar1"]["w"], params["proj"]["radar1"]["b"]),
        "radar2": _proj(x_radar2, params["proj"]["radar2"]["w"], params["proj"]["radar2"]["b"]),
    }
    for layer in params["layers"]:
        c = layer["conv"]
        o_ll = _gatv2_sc(c["ll"], x["lidar"], x["lidar"], ei_ll, ea_ll)
        o_r1r1 = _gatv2_sc(c["r1r1"], x["radar1"], x["radar1"], ei_r1r1, ea_r1r1)
        o_lr1 = _gatv2_sc(c["lr1"], x["lidar"], x["radar1"], ei_lr1, ea_lr1)
        o_r2r2 = _gatv2_sc(c["r2r2"], x["radar2"], x["radar2"], ei_r2r2, ea_r2r2)
        o_lr2 = _gatv2_sc(c["lr2"], x["lidar"], x["radar2"], ei_lr2, ea_lr2)
        bn = layer["bn"]
        x = {
            "lidar": _bn(o_ll, None, c["ll"]["bias"], bn["lidar"]["g"], bn["lidar"]["b"]),
            "radar1": _bn(o_r1r1, o_lr1, c["r1r1"]["bias"] + c["lr1"]["bias"],
                          bn["radar1"]["g"], bn["radar1"]["b"]),
            "radar2": _bn(o_r2r2, o_lr2, c["r2r2"]["bias"] + c["lr2"]["bias"],
                          bn["radar2"]["g"], bn["radar2"]["b"]),
        }
    out_l = _head(x["lidar"], params["head_lidar"]["w"], params["head_lidar"]["b"])
    out_r1 = _head(x["radar1"], params["head_radar"]["w"], params["head_radar"]["b"])
    out_r2 = _head(x["radar2"], params["head_radar"]["w"], params["head_radar"]["b"])
    return (out_l, out_r1, out_r2)


# speedup vs baseline: 24.2289x; 1.2888x over previous
import functools
import jax
import jax.numpy as jnp
from jax import lax
from jax.experimental import pallas as pl
from jax.experimental.pallas import tpu as pltpu
from jax.experimental.pallas import tpu_sc as plsc

N = 10000
E = 160000
HID = 256
HALF = 128
B = 80
EPT = E // 16
NBLK = EPT // B
NEG = -1e30


def _sc_body(xl_hbm, xr_hbm, ee_hbm, src_hbm, dst_hbm, att_hbm, out_hbm, al_hbm,
             isrc, idst, igs, ixr, idx0, idx1, idx2, idx3,
             exb0, exb1, exb2, exb3,
             xlb, xrb, eeb, alb2, alb2b, dband, attb, mxst, mxrd,
             acc, den_sp, maxstage, sem0, sem1, sem2, sem3):
    exb4 = [exb0, exb1, exb2, exb3]
    c = lax.axis_index("c")
    s = lax.axis_index("s")
    cbase = c * N
    ebase = s * EPT
    zv = jnp.zeros((16,), jnp.float32)

    def zrow(i, _):
        for m in range(8):
            xlb[i, pl.ds(16 * m, 16)] = zv
        return 0
    lax.fori_loop(0, B, zrow, 0)
    for u in range(4):
        def zdb(i, _):
            dband[u, pl.ds(i * 16, 16)] = zv
            return 0
        lax.fori_loop(0, 40, zdb, 0)
    pltpu.sync_copy(att_hbm.at[pl.ds(c * HALF, HALF)], attb)

    nchunk = jnp.where(s < 15, 8, 5)
    def zchunk(k, _):
        pltpu.sync_copy(xlb, acc.at[pl.ds(pl.multiple_of(s * 640 + k * 80, 8), 80)])
        return 0
    lax.fori_loop(0, nchunk, zchunk, 0)
    for u in range(4):
        pltpu.sync_copy(dband.at[u],
                        den_sp.at[pl.ds(pl.multiple_of(s * 2560 + u * 640, 128), 640)])
    plsc.subcore_barrier()

    lanes = lax.broadcasted_iota(jnp.int32, (16,), 0)

    def p1_blk(j, mv):
        base = ebase + j * B
        pltpu.sync_copy(src_hbm.at[pl.ds(base, B)], isrc)
        pltpu.sync_copy(dst_hbm.at[pl.ds(base, B)], idst)
        for t in range(5):
            igs[pl.ds(16 * t, 16)] = isrc[pl.ds(16 * t, 16)] + cbase
            ixr[pl.ds(16 * t, 16)] = idst[pl.ds(16 * t, 16)] + cbase
        cpa = pltpu.async_copy(xl_hbm.at[igs], xlb, sem0)
        cpb = pltpu.async_copy(xr_hbm.at[ixr], xrb, sem1)
        cpc = pltpu.async_copy(ee_hbm.at[pl.ds(c * E + base, B)], eeb, sem2)
        cpa.wait()
        cpb.wait()
        cpc.wait()

        for t in range(5):
            def pedge(i, carry):
                e = 16 * t + i
                outs = []
                for k in range(4):
                    z0 = xlb[e, pl.ds(32 * k, 16)] + xrb[e, pl.ds(32 * k, 16)] + eeb[e, pl.ds(32 * k, 16)]
                    z1 = xlb[e, pl.ds(32 * k + 16, 16)] + xrb[e, pl.ds(32 * k + 16, 16)] + eeb[e, pl.ds(32 * k + 16, 16)]
                    h0 = jnp.maximum(z0, 0.2 * z0)
                    h1 = jnp.maximum(z1, 0.2 * z1)
                    ts = h0 * attb[pl.ds(32 * k, 16)] + h1 * attb[pl.ds(32 * k + 16, 16)]
                    for st in (1, 2, 4, 8):
                        ts = ts + ts.at[lanes ^ st].get(mode="promise_in_bounds")
                    outs.append(jnp.where(lanes == i, ts, carry[k]))
                return tuple(outs)
            a = lax.fori_loop(0, 16, pedge, (zv, zv, zv, zv))
            for k in range(4):
                alb2[k, pl.ds(16 * t, 16)] = a[k]
                mv = jnp.maximum(mv, a[k])
        pltpu.sync_copy(alb2, al_hbm.at[c].at[s].at[j])
        return mv
    mv = lax.fori_loop(0, NBLK, p1_blk, jnp.full((16,), NEG, jnp.float32))

    for m in range(8):
        mxst[pl.ds(16 * m, 16)] = mv
    pltpu.sync_copy(mxst, maxstage.at[s])
    plsc.subcore_barrier()
    pltpu.sync_copy(maxstage, mxrd)
    gv = mxrd[0, pl.ds(0, 16)]
    for t in range(1, 16):
        gv = jnp.maximum(gv, mxrd[t, pl.ds(0, 16)])
    for st in (1, 2, 4, 8):
        gv = jnp.maximum(gv, gv.at[lanes ^ st].get(mode="promise_in_bounds"))
    G = gv

    NP = 10240

    def p2_start(j, X, IS, IG, ID, AL, gsem, asem):
        base = ebase + j * B
        pltpu.sync_copy(src_hbm.at[pl.ds(base, B)], IS)
        pltpu.sync_copy(dst_hbm.at[pl.ds(base, B)], ID)
        for t in range(5):
            IG[pl.ds(16 * t, 16)] = IS[pl.ds(16 * t, 16)] + cbase
        pltpu.async_copy(xl_hbm.at[IG], X, gsem)
        pltpu.async_copy(al_hbm.at[c].at[s].at[j], AL, asem)

    def p2_finish(X, ID, AL, gsem, asem):
        pltpu.make_async_copy(al_hbm.at[0].at[0].at[0], AL, asem).wait()
        for k in range(4):
            for t in range(5):
                exb4[k][pl.ds(16 * t, 16)] = jnp.exp(AL[k, pl.ds(16 * t, 16)] - G)
        for k in range(4):
            for t in range(5):
                idx2[pl.ds(16 * t, 16)] = ID[pl.ds(16 * t, 16)] + (NP * k)
            pltpu.sync_copy(exb4[k], den_sp.at[idx2], add=True)
        pltpu.make_async_copy(xl_hbm.at[pl.ds(0, B)], X, gsem).wait()
        for t in range(5):
            a16 = [exb4[k][pl.ds(16 * t, 16)] for k in range(4)]

            def pedge(i, _):
                e = 16 * t + i
                sel = lanes * 0 + i
                for k in range(4):
                    av = a16[k].at[sel].get(mode="promise_in_bounds")
                    X[e, pl.ds(32 * k, 16)] = X[e, pl.ds(32 * k, 16)] * av
                    X[e, pl.ds(32 * k + 16, 16)] = X[e, pl.ds(32 * k + 16, 16)] * av
                return 0
            lax.fori_loop(0, 16, pedge, 0)
        pltpu.sync_copy(X, acc.at[ID], add=True)

    setA = (xlb, isrc, igs, idst, alb2, sem0, sem2)
    setB = (eeb, ixr, idx0, idx1, alb2b, sem1, sem3)

    def _start(j, S):
        p2_start(j, S[0], S[1], S[2], S[3], S[4], S[5], S[6])

    def _finish(S):
        p2_finish(S[0], S[3], S[4], S[5], S[6])

    _start(0, setA)

    def p2_pair(m, _):
        _start(2 * m + 1, setB)
        _finish(setA)
        _start(2 * m + 2, setA)
        _finish(setB)
        return 0
    lax.fori_loop(0, (NBLK - 1) // 2, p2_pair, 0)
    _finish(setA)
    plsc.subcore_barrier()

    for k in range(4):
        pltpu.sync_copy(den_sp.at[pl.ds(pl.multiple_of(NP * k + s * 640, 128), 640)],
                        dband.at[k])
    for k in range(4):
        for t in range(40):
            dband[k, pl.ds(16 * t, 16)] = 1.0 / (dband[k, pl.ds(16 * t, 16)] + 1e-16)
    nv = jnp.where(s < 15, 8, 5)
    for v in range(8):
        @pl.when(v < nv)
        def _():
            pltpu.sync_copy(
                acc.at[pl.ds(pl.multiple_of(s * 640 + v * 80, 8), 80)], xlb)
            for g in range(5):
                rb = [dband[k, pl.ds(16 * (5 * v + g), 16)] for k in range(4)]

                def prow(i, _):
                    e = 16 * g + i
                    sel = lanes * 0 + i
                    for k in range(4):
                        av = rb[k].at[sel].get(mode="promise_in_bounds")
                        xlb[e, pl.ds(32 * k, 16)] = xlb[e, pl.ds(32 * k, 16)] * av
                        xlb[e, pl.ds(32 * k + 16, 16)] = xlb[e, pl.ds(32 * k + 16, 16)] * av
                    return 0
                lax.fori_loop(0, 16, prow, 0)
            pltpu.sync_copy(
                xlb, out_hbm.at[pl.ds(pl.multiple_of(cbase + s * 640 + v * 80, 8), 80)])


@jax.jit
def _sc_edge(xl_cat, xr_cat, ee_cat, src, dst, att_flat):
    mesh = plsc.VectorSubcoreMesh(core_axis_name="c", subcore_axis_name="s")
    f = pl.kernel(
        _sc_body,
        out_type=[
            jax.ShapeDtypeStruct((2 * N, HALF), jnp.float32),
            jax.ShapeDtypeStruct((2, 16, NBLK, 4, B), jnp.float32),
        ],
        mesh=mesh,
        scratch_types=(
            [pltpu.VMEM((B,), jnp.int32)] * 4
            + [pltpu.VMEM((B,), jnp.int32)] * 4
            + [pltpu.VMEM((B,), jnp.float32)] * 4
            + [
                pltpu.VMEM((B, HALF), jnp.float32),
                pltpu.VMEM((B, HALF), jnp.float32),
                pltpu.VMEM((B, HALF), jnp.float32),
                pltpu.VMEM((4, B), jnp.float32),
                pltpu.VMEM((4, B), jnp.float32),
                pltpu.VMEM((4, 640), jnp.float32),
                pltpu.VMEM((HALF,), jnp.float32),
                pltpu.VMEM((HALF,), jnp.float32),
                pltpu.VMEM((16, HALF), jnp.float32),
                pltpu.VMEM_SHARED((N, HALF), jnp.float32),
                pltpu.VMEM_SHARED((40960,), jnp.float32),
                pltpu.VMEM_SHARED((16, HALF), jnp.float32),
                pltpu.SemaphoreType.DMA,
                pltpu.SemaphoreType.DMA,
                pltpu.SemaphoreType.DMA,
                pltpu.SemaphoreType.DMA,
            ]
        ),
    )
    return f(xl_cat, xr_cat, ee_cat, src, dst, att_flat)[0]


def _proj_body(x_ref, w_ref, b_ref, o_ref):
    y = jnp.dot(x_ref[...], w_ref[0], preferred_element_type=jnp.float32)
    y = y + b_ref[0, 0, :]
    o_ref[0] = jnp.where(y > 0, y, jnp.exp(jnp.minimum(y, 0.0)) - 1.0)


def _proj(x, w, b):
    m, k = x.shape
    blk = 2000
    return pl.pallas_call(
        _proj_body,
        grid=(m // blk, 2),
        in_specs=[
            pl.BlockSpec((blk, k), lambda i, h: (i, 0)),
            pl.BlockSpec((1, k, HALF), lambda i, h: (h, 0, 0)),
            pl.BlockSpec((1, 1, HALF), lambda i, h: (h, 0, 0)),
        ],
        out_specs=pl.BlockSpec((1, blk, HALF), lambda i, h: (h, i, 0)),
        out_shape=jax.ShapeDtypeStruct((2, m, HALF), jnp.float32),
    )(x, jnp.transpose(w.reshape(k, 2, HALF), (1, 0, 2)), b.reshape(2, 1, HALF))


def _lin_body(x_ref, w_ref, b_ref, o_ref):
    y = jnp.dot(x_ref[0], w_ref[0, 0], preferred_element_type=jnp.float32)
    y = y + jnp.dot(x_ref[1], w_ref[0, 1], preferred_element_type=jnp.float32)
    o_ref[0] = y + b_ref[0, 0, :]


def _lin_split(x2, w, b):
    m = x2.shape[1]
    blk = 2000
    return pl.pallas_call(
        _lin_body,
        grid=(m // blk, 2),
        in_specs=[
            pl.BlockSpec((2, blk, HALF), lambda i, h: (0, i, 0)),
            pl.BlockSpec((1, 2, HALF, HALF), lambda i, h: (h, 0, 0, 0)),
            pl.BlockSpec((1, 1, HALF), lambda i, h: (h, 0, 0)),
        ],
        out_specs=pl.BlockSpec((1, blk, HALF), lambda i, h: (h, i, 0)),
        out_shape=jax.ShapeDtypeStruct((2, m, HALF), jnp.float32),
    )(x2, jnp.transpose(w.reshape(2, HALF, 2, HALF), (2, 0, 1, 3)),
      b.reshape(2, 1, HALF))


def _ee_body(ea_ref, we_ref, o_ref):
    ea = ea_ref[...]
    y = ea[:, 0][:, None] * we_ref[0, 0, :][None, :]
    for j in range(1, 4):
        y = y + ea[:, j][:, None] * we_ref[0, j, :][None, :]
    o_ref[0] = y


def _ee(ea, we):
    m = ea.shape[0]
    blk = 2000
    return pl.pallas_call(
        _ee_body,
        grid=(m // blk, 2),
        in_specs=[
            pl.BlockSpec((blk, 4), lambda i, h: (i, 0)),
            pl.BlockSpec((1, 4, HALF), lambda i, h: (h, 0, 0)),
        ],
        out_specs=pl.BlockSpec((1, blk, HALF), lambda i, h: (h, i, 0)),
        out_shape=jax.ShapeDtypeStruct((2, m, HALF), jnp.float32),
    )(ea, jnp.transpose(we.reshape(4, 2, HALF), (1, 0, 2)))


def _bn1_body(o1_ref, bias_ref, g_ref, b_ref, o_ref):
    x = o1_ref[0] + bias_ref[0, 0, :]
    y = jnp.where(x > 0, x, jnp.exp(jnp.minimum(x, 0.0)) - 1.0)
    m = jnp.mean(y, axis=0)
    v = jnp.mean((y - m[None, :]) ** 2, axis=0)
    o_ref[0] = (y - m[None, :]) * jax.lax.rsqrt(v + 1e-5)[None, :] * g_ref[0, 0, :] + b_ref[0, 0, :]


def _bn2_body(o1_ref, o2_ref, bias_ref, g_ref, b_ref, o_ref):
    x = o1_ref[0] + o2_ref[0] + bias_ref[0, 0, :]
    y = jnp.where(x > 0, x, jnp.exp(jnp.minimum(x, 0.0)) - 1.0)
    m = jnp.mean(y, axis=0)
    v = jnp.mean((y - m[None, :]) ** 2, axis=0)
    o_ref[0] = (y - m[None, :]) * jax.lax.rsqrt(v + 1e-5)[None, :] * g_ref[0, 0, :] + b_ref[0, 0, :]


def _bn(o1, o2, bias, g, b):
    vec = pl.BlockSpec((1, 1, HALF), lambda h: (h, 0, 0))
    full = pl.BlockSpec((1, N, HALF), lambda h: (h, 0, 0))
    if o2 is None:
        return pl.pallas_call(
            _bn1_body, grid=(2,),
            in_specs=[full, vec, vec, vec],
            out_specs=full,
            out_shape=jax.ShapeDtypeStruct((2, N, HALF), jnp.float32),
        )(o1, bias.reshape(2, 1, HALF), g.reshape(2, 1, HALF), b.reshape(2, 1, HALF))
    return pl.pallas_call(
        _bn2_body, grid=(2,),
        in_specs=[full, full, vec, vec, vec],
        out_specs=full,
        out_shape=jax.ShapeDtypeStruct((2, N, HALF), jnp.float32),
    )(o1, o2, bias.reshape(2, 1, HALF), g.reshape(2, 1, HALF), b.reshape(2, 1, HALF))


def _head_body(x_ref, w_ref, b_ref, o_ref):
    y = jnp.dot(x_ref[0], w_ref[0][:, None], preferred_element_type=jnp.float32)
    y = y + jnp.dot(x_ref[1], w_ref[1][:, None], preferred_element_type=jnp.float32)
    o_ref[...] = y + b_ref[0]


def _head(x2, w, b):
    blk = 2000
    return pl.pallas_call(
        _head_body,
        grid=(N // blk,),
        in_specs=[
            pl.BlockSpec((2, blk, HALF), lambda i: (0, i, 0)),
            pl.BlockSpec((2, HALF), lambda i: (0, 0)),
            pl.BlockSpec((1,), lambda i: (0,)),
        ],
        out_specs=pl.BlockSpec((blk, 1), lambda i: (i, 0)),
        out_shape=jax.ShapeDtypeStruct((N, 1), jnp.float32),
    )(x2, w.reshape(2, HALF), b)


def _gatv2_sc(p, x_src2, x_dst2, ei, ea):
    xl = _lin_split(x_src2, p["wl"], p["bl"])
    xr = _lin_split(x_dst2, p["wr"], p["br"])
    ee = _ee(ea, p["we"])
    o = _sc_edge(xl.reshape(2 * N, HALF), xr.reshape(2 * N, HALF),
                 ee.reshape(2 * E, HALF), ei[0], ei[1],
                 p["att"].reshape(HID))
    return o.reshape(2, N, HALF)


def kernel(x_lidar, x_radar1, x_radar2, ei_ll, ei_r1r1, ei_r2r2, ei_lr1, ei_lr2,
           ea_ll, ea_r1r1, ea_r2r2, ea_lr1, ea_lr2, params):
    x = {
        "lidar": _proj(x_lidar, params["proj"]["lidar"]["w"], params["proj"]["lidar"]["b"]),
        "radar1": _proj(x_radar1, params["proj"]["radar1"]["w"], params["proj"]["radar1"]["b"]),
        "radar2": _proj(x_radar2, params["proj"]["radar2"]["w"], params["proj"]["radar2"]["b"]),
    }
    for layer in params["layers"]:
        c = layer["conv"]
        o_ll = _gatv2_sc(c["ll"], x["lidar"], x["lidar"], ei_ll, ea_ll)
        o_r1r1 = _gatv2_sc(c["r1r1"], x["radar1"], x["radar1"], ei_r1r1, ea_r1r1)
        o_lr1 = _gatv2_sc(c["lr1"], x["lidar"], x["radar1"], ei_lr1, ea_lr1)
        o_r2r2 = _gatv2_sc(c["r2r2"], x["radar2"], x["radar2"], ei_r2r2, ea_r2r2)
        o_lr2 = _gatv2_sc(c["lr2"], x["lidar"], x["radar2"], ei_lr2, ea_lr2)
        bn = layer["bn"]
        x = {
            "lidar": _bn(o_ll, None, c["ll"]["bias"], bn["lidar"]["g"], bn["lidar"]["b"]),
            "radar1": _bn(o_r1r1, o_lr1, c["r1r1"]["bias"] + c["lr1"]["bias"],
                          bn["radar1"]["g"], bn["radar1"]["b"]),
            "radar2": _bn(o_r2r2, o_lr2, c["r2r2"]["bias"] + c["lr2"]["bias"],
                          bn["radar2"]["g"], bn["radar2"]["b"]),
        }
    out_l = _head(x["lidar"], params["head_lidar"]["w"], params["head_lidar"]["b"])
    out_r1 = _head(x["radar1"], params["head_radar"]["w"], params["head_radar"]["b"])
    out_r2 = _head(x["radar2"], params["head_radar"]["w"], params["head_radar"]["b"])
    return (out_l, out_r1, out_r2)


# speedup vs baseline: 26.9533x; 1.1124x over previous
import functools
import jax
import jax.numpy as jnp
from jax import lax
from jax.experimental import pallas as pl
from jax.experimental.pallas import tpu as pltpu
from jax.experimental.pallas import tpu_sc as plsc

N = 10000
E = 160000
HID = 256
HALF = 128
B = 80
EPT = E // 16
NBLK = EPT // B
NEG = -1e30


def _sc_body(xl_hbm, xr_hbm, ee_hbm, src_hbm, dst_hbm, att_hbm, out_hbm, al_hbm,
             isrc, idst, igs, ixr, idx0, idx1, idx2, idx3,
             exb0, exb1, exb2, exb3,
             xlb, xrb, eeb, alb2, alb2b, dband, attb, mxst, mxrd,
             acc, den_sp, maxstage, sem0, sem1, sem2, sem3, sem4, sem5):
    exb4 = [exb0, exb1, exb2, exb3]
    c = lax.axis_index("c")
    s = lax.axis_index("s")
    cbase = c * N
    ebase = s * EPT
    zv = jnp.zeros((16,), jnp.float32)

    def zrow(i, _):
        for m in range(8):
            xlb[i, pl.ds(16 * m, 16)] = zv
        return 0
    lax.fori_loop(0, B, zrow, 0)
    for u in range(4):
        def zdb(i, _):
            dband[u, pl.ds(i * 16, 16)] = zv
            return 0
        lax.fori_loop(0, 40, zdb, 0)
    pltpu.sync_copy(att_hbm.at[pl.ds(c * HALF, HALF)], attb)

    nchunk = jnp.where(s < 15, 8, 5)
    def zchunk(k, _):
        pltpu.sync_copy(xlb, acc.at[pl.ds(pl.multiple_of(s * 640 + k * 80, 8), 80)])
        return 0
    lax.fori_loop(0, nchunk, zchunk, 0)
    for u in range(4):
        pltpu.sync_copy(dband.at[u],
                        den_sp.at[pl.ds(pl.multiple_of(s * 2560 + u * 640, 128), 640)])
    plsc.subcore_barrier()

    lanes = lax.broadcasted_iota(jnp.int32, (16,), 0)

    def p1_step(j, mv, IS, ID, IG, IX, AL, semg, semi, semw):
        base = ebase + j * B
        pltpu.make_async_copy(src_hbm.at[pl.ds(0, B)], IS, semi).wait()
        pltpu.make_async_copy(src_hbm.at[pl.ds(0, B)], ID, semi).wait()
        for t in range(5):
            IG[pl.ds(16 * t, 16)] = IS[pl.ds(16 * t, 16)] + cbase
            IX[pl.ds(16 * t, 16)] = ID[pl.ds(16 * t, 16)] + cbase
        cpa = pltpu.async_copy(xl_hbm.at[IG], xlb, semg)
        cpb = pltpu.async_copy(xr_hbm.at[IX], xrb, semg)
        cpc = pltpu.async_copy(ee_hbm.at[pl.ds(c * E + base, B)], eeb, semg)
        b2 = ebase + jnp.minimum(j + 2, NBLK - 1) * B
        pltpu.async_copy(src_hbm.at[pl.ds(b2, B)], IS, semi)
        pltpu.async_copy(dst_hbm.at[pl.ds(b2, B)], ID, semi)
        cpa.wait()
        cpb.wait()
        cpc.wait()

        @pl.when(j >= 2)
        def _():
            pltpu.make_async_copy(al_hbm.at[0].at[0].at[0], AL, semw).wait()

        for t in range(5):
            def pedge(i, carry):
                e = 16 * t + i
                outs = []
                for k in range(4):
                    z0 = xlb[e, pl.ds(32 * k, 16)] + xrb[e, pl.ds(32 * k, 16)] + eeb[e, pl.ds(32 * k, 16)]
                    z1 = xlb[e, pl.ds(32 * k + 16, 16)] + xrb[e, pl.ds(32 * k + 16, 16)] + eeb[e, pl.ds(32 * k + 16, 16)]
                    h0 = jnp.maximum(z0, 0.2 * z0)
                    h1 = jnp.maximum(z1, 0.2 * z1)
                    ts = h0 * attb[pl.ds(32 * k, 16)] + h1 * attb[pl.ds(32 * k + 16, 16)]
                    for st in (1, 2, 4, 8):
                        ts = ts + ts.at[lanes ^ st].get(mode="promise_in_bounds")
                    outs.append(jnp.where(lanes == i, ts, carry[k]))
                return tuple(outs)
            a = lax.fori_loop(0, 16, pedge, (zv, zv, zv, zv))
            for k in range(4):
                AL[k, pl.ds(16 * t, 16)] = a[k]
                mv = jnp.maximum(mv, a[k])
        pltpu.async_copy(AL, al_hbm.at[c].at[s].at[j], semw)
        return mv

    pA = (isrc, idst, igs, ixr, alb2, sem0, sem2, sem4)
    pB = (idx0, idx1, idx2, idx3, alb2b, sem1, sem3, sem5)
    pltpu.async_copy(src_hbm.at[pl.ds(ebase, B)], isrc, sem2)
    pltpu.async_copy(dst_hbm.at[pl.ds(ebase, B)], idst, sem2)
    pltpu.async_copy(src_hbm.at[pl.ds(ebase + B, B)], idx0, sem3)
    pltpu.async_copy(dst_hbm.at[pl.ds(ebase + B, B)], idx1, sem3)

    def p1_pair(m, mv):
        mv = p1_step(2 * m, mv, *pA)
        mv = p1_step(2 * m + 1, mv, *pB)
        return mv
    mv = lax.fori_loop(0, (NBLK - 1) // 2, p1_pair,
                       jnp.full((16,), NEG, jnp.float32))
    mv = p1_step(NBLK - 1, mv, *pA)
    pltpu.make_async_copy(al_hbm.at[0].at[0].at[0], alb2, sem4).wait()
    pltpu.make_async_copy(al_hbm.at[0].at[0].at[0], alb2b, sem5).wait()
    for IS, ID, semi in ((isrc, idst, sem2), (idx0, idx1, sem3)):
        pltpu.make_async_copy(src_hbm.at[pl.ds(0, B)], IS, semi).wait()
        pltpu.make_async_copy(src_hbm.at[pl.ds(0, B)], ID, semi).wait()

    for m in range(8):
        mxst[pl.ds(16 * m, 16)] = mv
    pltpu.sync_copy(mxst, maxstage.at[s])
    plsc.subcore_barrier()
    pltpu.sync_copy(maxstage, mxrd)
    gv = mxrd[0, pl.ds(0, 16)]
    for t in range(1, 16):
        gv = jnp.maximum(gv, mxrd[t, pl.ds(0, 16)])
    for st in (1, 2, 4, 8):
        gv = jnp.maximum(gv, gv.at[lanes ^ st].get(mode="promise_in_bounds"))
    G = gv

    NP = 10240

    def p2_start(j, X, IS, IG, ID, AL, gsem, asem):
        base = ebase + j * B
        pltpu.sync_copy(src_hbm.at[pl.ds(base, B)], IS)
        pltpu.sync_copy(dst_hbm.at[pl.ds(base, B)], ID)
        for t in range(5):
            IG[pl.ds(16 * t, 16)] = IS[pl.ds(16 * t, 16)] + cbase
        pltpu.async_copy(xl_hbm.at[IG], X, gsem)
        pltpu.async_copy(al_hbm.at[c].at[s].at[j], AL, asem)

    def p2_finish(X, ID, AL, gsem, asem):
        pltpu.make_async_copy(al_hbm.at[0].at[0].at[0], AL, asem).wait()
        for k in range(4):
            for t in range(5):
                exb4[k][pl.ds(16 * t, 16)] = jnp.exp(AL[k, pl.ds(16 * t, 16)] - G)
        for k in range(4):
            for t in range(5):
                idx2[pl.ds(16 * t, 16)] = ID[pl.ds(16 * t, 16)] + (NP * k)
            pltpu.sync_copy(exb4[k], den_sp.at[idx2], add=True)
        pltpu.make_async_copy(xl_hbm.at[pl.ds(0, B)], X, gsem).wait()
        for t in range(5):
            a16 = [exb4[k][pl.ds(16 * t, 16)] for k in range(4)]

            def pedge(i, _):
                e = 16 * t + i
                sel = lanes * 0 + i
                for k in range(4):
                    av = a16[k].at[sel].get(mode="promise_in_bounds")
                    X[e, pl.ds(32 * k, 16)] = X[e, pl.ds(32 * k, 16)] * av
                    X[e, pl.ds(32 * k + 16, 16)] = X[e, pl.ds(32 * k + 16, 16)] * av
                return 0
            lax.fori_loop(0, 16, pedge, 0)
        pltpu.sync_copy(X, acc.at[ID], add=True)

    setA = (xlb, isrc, igs, idst, alb2, sem0, sem2)
    setB = (eeb, ixr, idx0, idx1, alb2b, sem1, sem3)

    def _start(j, S):
        p2_start(j, S[0], S[1], S[2], S[3], S[4], S[5], S[6])

    def _finish(S):
        p2_finish(S[0], S[3], S[4], S[5], S[6])

    _start(0, setA)

    def p2_pair(m, _):
        _start(2 * m + 1, setB)
        _finish(setA)
        _start(2 * m + 2, setA)
        _finish(setB)
        return 0
    lax.fori_loop(0, (NBLK - 1) // 2, p2_pair, 0)
    _finish(setA)
    plsc.subcore_barrier()

    for k in range(4):
        pltpu.sync_copy(den_sp.at[pl.ds(pl.multiple_of(NP * k + s * 640, 128), 640)],
                        dband.at[k])
    for k in range(4):
        for t in range(40):
            dband[k, pl.ds(16 * t, 16)] = 1.0 / (dband[k, pl.ds(16 * t, 16)] + 1e-16)
    nv = jnp.where(s < 15, 8, 5)
    for v in range(8):
        @pl.when(v < nv)
        def _():
            pltpu.sync_copy(
                acc.at[pl.ds(pl.multiple_of(s * 640 + v * 80, 8), 80)], xlb)
            for g in range(5):
                rb = [dband[k, pl.ds(16 * (5 * v + g), 16)] for k in range(4)]

                def prow(i, _):
                    e = 16 * g + i
                    sel = lanes * 0 + i
                    for k in range(4):
                        av = rb[k].at[sel].get(mode="promise_in_bounds")
                        xlb[e, pl.ds(32 * k, 16)] = xlb[e, pl.ds(32 * k, 16)] * av
                        xlb[e, pl.ds(32 * k + 16, 16)] = xlb[e, pl.ds(32 * k + 16, 16)] * av
                    return 0
                lax.fori_loop(0, 16, prow, 0)
            pltpu.sync_copy(
                xlb, out_hbm.at[pl.ds(pl.multiple_of(cbase + s * 640 + v * 80, 8), 80)])


@jax.jit
def _sc_edge(xl_cat, xr_cat, ee_cat, src, dst, att_flat):
    mesh = plsc.VectorSubcoreMesh(core_axis_name="c", subcore_axis_name="s")
    f = pl.kernel(
        _sc_body,
        out_type=[
            jax.ShapeDtypeStruct((2 * N, HALF), jnp.float32),
            jax.ShapeDtypeStruct((2, 16, NBLK, 4, B), jnp.float32),
        ],
        mesh=mesh,
        scratch_types=(
            [pltpu.VMEM((B,), jnp.int32)] * 4
            + [pltpu.VMEM((B,), jnp.int32)] * 4
            + [pltpu.VMEM((B,), jnp.float32)] * 4
            + [
                pltpu.VMEM((B, HALF), jnp.float32),
                pltpu.VMEM((B, HALF), jnp.float32),
                pltpu.VMEM((B, HALF), jnp.float32),
                pltpu.VMEM((4, B), jnp.float32),
                pltpu.VMEM((4, B), jnp.float32),
                pltpu.VMEM((4, 640), jnp.float32),
                pltpu.VMEM((HALF,), jnp.float32),
                pltpu.VMEM((HALF,), jnp.float32),
                pltpu.VMEM((16, HALF), jnp.float32),
                pltpu.VMEM_SHARED((N, HALF), jnp.float32),
                pltpu.VMEM_SHARED((40960,), jnp.float32),
                pltpu.VMEM_SHARED((16, HALF), jnp.float32),
                pltpu.SemaphoreType.DMA,
                pltpu.SemaphoreType.DMA,
                pltpu.SemaphoreType.DMA,
                pltpu.SemaphoreType.DMA,
                pltpu.SemaphoreType.DMA,
                pltpu.SemaphoreType.DMA,
            ]
        ),
    )
    return f(xl_cat, xr_cat, ee_cat, src, dst, att_flat)[0]


def _proj_body(x_ref, w_ref, b_ref, o_ref):
    y = jnp.dot(x_ref[...], w_ref[0], preferred_element_type=jnp.float32)
    y = y + b_ref[0, 0, :]
    o_ref[0] = jnp.where(y > 0, y, jnp.exp(jnp.minimum(y, 0.0)) - 1.0)


def _proj(x, w, b):
    m, k = x.shape
    blk = 2000
    return pl.pallas_call(
        _proj_body,
        grid=(m // blk, 2),
        in_specs=[
            pl.BlockSpec((blk, k), lambda i, h: (i, 0)),
            pl.BlockSpec((1, k, HALF), lambda i, h: (h, 0, 0)),
            pl.BlockSpec((1, 1, HALF), lambda i, h: (h, 0, 0)),
        ],
        out_specs=pl.BlockSpec((1, blk, HALF), lambda i, h: (h, i, 0)),
        out_shape=jax.ShapeDtypeStruct((2, m, HALF), jnp.float32),
    )(x, jnp.transpose(w.reshape(k, 2, HALF), (1, 0, 2)), b.reshape(2, 1, HALF))


def _lin_body(x_ref, w_ref, b_ref, o_ref):
    y = jnp.dot(x_ref[0], w_ref[0, 0], preferred_element_type=jnp.float32)
    y = y + jnp.dot(x_ref[1], w_ref[0, 1], preferred_element_type=jnp.float32)
    o_ref[0] = y + b_ref[0, 0, :]


def _lin_split(x2, w, b):
    m = x2.shape[1]
    blk = 2000
    return pl.pallas_call(
        _lin_body,
        grid=(m // blk, 2),
        in_specs=[
            pl.BlockSpec((2, blk, HALF), lambda i, h: (0, i, 0)),
            pl.BlockSpec((1, 2, HALF, HALF), lambda i, h: (h, 0, 0, 0)),
            pl.BlockSpec((1, 1, HALF), lambda i, h: (h, 0, 0)),
        ],
        out_specs=pl.BlockSpec((1, blk, HALF), lambda i, h: (h, i, 0)),
        out_shape=jax.ShapeDtypeStruct((2, m, HALF), jnp.float32),
    )(x2, jnp.transpose(w.reshape(2, HALF, 2, HALF), (2, 0, 1, 3)),
      b.reshape(2, 1, HALF))


def _ee_body(ea_ref, we_ref, o_ref):
    ea = ea_ref[...]
    y = ea[:, 0][:, None] * we_ref[0, 0, :][None, :]
    for j in range(1, 4):
        y = y + ea[:, j][:, None] * we_ref[0, j, :][None, :]
    o_ref[0] = y


def _ee(ea, we):
    m = ea.shape[0]
    blk = 2000
    return pl.pallas_call(
        _ee_body,
        grid=(m // blk, 2),
        in_specs=[
            pl.BlockSpec((blk, 4), lambda i, h: (i, 0)),
            pl.BlockSpec((1, 4, HALF), lambda i, h: (h, 0, 0)),
        ],
        out_specs=pl.BlockSpec((1, blk, HALF), lambda i, h: (h, i, 0)),
        out_shape=jax.ShapeDtypeStruct((2, m, HALF), jnp.float32),
    )(ea, jnp.transpose(we.reshape(4, 2, HALF), (1, 0, 2)))


def _bn1_body(o1_ref, bias_ref, g_ref, b_ref, o_ref):
    x = o1_ref[0] + bias_ref[0, 0, :]
    y = jnp.where(x > 0, x, jnp.exp(jnp.minimum(x, 0.0)) - 1.0)
    m = jnp.mean(y, axis=0)
    v = jnp.mean((y - m[None, :]) ** 2, axis=0)
    o_ref[0] = (y - m[None, :]) * jax.lax.rsqrt(v + 1e-5)[None, :] * g_ref[0, 0, :] + b_ref[0, 0, :]


def _bn2_body(o1_ref, o2_ref, bias_ref, g_ref, b_ref, o_ref):
    x = o1_ref[0] + o2_ref[0] + bias_ref[0, 0, :]
    y = jnp.where(x > 0, x, jnp.exp(jnp.minimum(x, 0.0)) - 1.0)
    m = jnp.mean(y, axis=0)
    v = jnp.mean((y - m[None, :]) ** 2, axis=0)
    o_ref[0] = (y - m[None, :]) * jax.lax.rsqrt(v + 1e-5)[None, :] * g_ref[0, 0, :] + b_ref[0, 0, :]


def _bn(o1, o2, bias, g, b):
    vec = pl.BlockSpec((1, 1, HALF), lambda h: (h, 0, 0))
    full = pl.BlockSpec((1, N, HALF), lambda h: (h, 0, 0))
    if o2 is None:
        return pl.pallas_call(
            _bn1_body, grid=(2,),
            in_specs=[full, vec, vec, vec],
            out_specs=full,
            out_shape=jax.ShapeDtypeStruct((2, N, HALF), jnp.float32),
        )(o1, bias.reshape(2, 1, HALF), g.reshape(2, 1, HALF), b.reshape(2, 1, HALF))
    return pl.pallas_call(
        _bn2_body, grid=(2,),
        in_specs=[full, full, vec, vec, vec],
        out_specs=full,
        out_shape=jax.ShapeDtypeStruct((2, N, HALF), jnp.float32),
    )(o1, o2, bias.reshape(2, 1, HALF), g.reshape(2, 1, HALF), b.reshape(2, 1, HALF))


def _head_body(x_ref, w_ref, b_ref, o_ref):
    y = jnp.dot(x_ref[0], w_ref[0][:, None], preferred_element_type=jnp.float32)
    y = y + jnp.dot(x_ref[1], w_ref[1][:, None], preferred_element_type=jnp.float32)
    o_ref[...] = y + b_ref[0]


def _head(x2, w, b):
    blk = 2000
    return pl.pallas_call(
        _head_body,
        grid=(N // blk,),
        in_specs=[
            pl.BlockSpec((2, blk, HALF), lambda i: (0, i, 0)),
            pl.BlockSpec((2, HALF), lambda i: (0, 0)),
            pl.BlockSpec((1,), lambda i: (0,)),
        ],
        out_specs=pl.BlockSpec((blk, 1), lambda i: (i, 0)),
        out_shape=jax.ShapeDtypeStruct((N, 1), jnp.float32),
    )(x2, w.reshape(2, HALF), b)


def _gatv2_sc(p, x_src2, x_dst2, ei, ea):
    xl = _lin_split(x_src2, p["wl"], p["bl"])
    xr = _lin_split(x_dst2, p["wr"], p["br"])
    ee = _ee(ea, p["we"])
    o = _sc_edge(xl.reshape(2 * N, HALF), xr.reshape(2 * N, HALF),
                 ee.reshape(2 * E, HALF), ei[0], ei[1],
                 p["att"].reshape(HID))
    return o.reshape(2, N, HALF)


def kernel(x_lidar, x_radar1, x_radar2, ei_ll, ei_r1r1, ei_r2r2, ei_lr1, ei_lr2,
           ea_ll, ea_r1r1, ea_r2r2, ea_lr1, ea_lr2, params):
    x = {
        "lidar": _proj(x_lidar, params["proj"]["lidar"]["w"], params["proj"]["lidar"]["b"]),
        "radar1": _proj(x_radar1, params["proj"]["radar1"]["w"], params["proj"]["radar1"]["b"]),
        "radar2": _proj(x_radar2, params["proj"]["radar2"]["w"], params["proj"]["radar2"]["b"]),
    }
    for layer in params["layers"]:
        c = layer["conv"]
        o_ll = _gatv2_sc(c["ll"], x["lidar"], x["lidar"], ei_ll, ea_ll)
        o_r1r1 = _gatv2_sc(c["r1r1"], x["radar1"], x["radar1"], ei_r1r1, ea_r1r1)
        o_lr1 = _gatv2_sc(c["lr1"], x["lidar"], x["radar1"], ei_lr1, ea_lr1)
        o_r2r2 = _gatv2_sc(c["r2r2"], x["radar2"], x["radar2"], ei_r2r2, ea_r2r2)
        o_lr2 = _gatv2_sc(c["lr2"], x["lidar"], x["radar2"], ei_lr2, ea_lr2)
        bn = layer["bn"]
        x = {
            "lidar": _bn(o_ll, None, c["ll"]["bias"], bn["lidar"]["g"], bn["lidar"]["b"]),
            "radar1": _bn(o_r1r1, o_lr1, c["r1r1"]["bias"] + c["lr1"]["bias"],
                          bn["radar1"]["g"], bn["radar1"]["b"]),
            "radar2": _bn(o_r2r2, o_lr2, c["r2r2"]["bias"] + c["lr2"]["bias"],
                          bn["radar2"]["g"], bn["radar2"]["b"]),
        }
    out_l = _head(x["lidar"], params["head_lidar"]["w"], params["head_lidar"]["b"])
    out_r1 = _head(x["radar1"], params["head_radar"]["w"], params["head_radar"]["b"])
    out_r2 = _head(x["radar2"], params["head_radar"]["w"], params["head_radar"]["b"])
    return (out_l, out_r1, out_r2)


# speedup vs baseline: 27.9021x; 1.0352x over previous
import functools
import jax
import jax.numpy as jnp
from jax import lax
from jax.experimental import pallas as pl
from jax.experimental.pallas import tpu as pltpu
from jax.experimental.pallas import tpu_sc as plsc

N = 10000
E = 160000
HID = 256
HALF = 128
B = 80
EPT = E // 16
NBLK = EPT // B
NEG = -1e30


def _sc_body(xl_hbm, xr_hbm, ee_hbm, src_hbm, dst_hbm, att_hbm, out_hbm, al_hbm,
             isrc, idst, igs, ixr, idx0, idx1, idx2, idx3,
             exb0, exb1, exb2, exb3,
             xlb, xrb, eeb, alb2, alb2b, dband, attb, mxst, mxrd,
             acc, den_sp, maxstage, sem0, sem1, sem2, sem3, sem4, sem5):
    exb4 = [exb0, exb1, exb2, exb3]
    c = lax.axis_index("c")
    s = lax.axis_index("s")
    cbase = c * N
    ebase = s * EPT
    zv = jnp.zeros((16,), jnp.float32)

    def zrow(i, _):
        for m in range(8):
            xlb[i, pl.ds(16 * m, 16)] = zv
        return 0
    lax.fori_loop(0, B, zrow, 0)
    for u in range(4):
        def zdb(i, _):
            dband[u, pl.ds(i * 16, 16)] = zv
            return 0
        lax.fori_loop(0, 40, zdb, 0)
    pltpu.sync_copy(att_hbm.at[pl.ds(c * HALF, HALF)], attb)

    nchunk = jnp.where(s < 15, 8, 5)
    def zchunk(k, _):
        pltpu.sync_copy(xlb, acc.at[pl.ds(pl.multiple_of(s * 640 + k * 80, 8), 80)])
        return 0
    lax.fori_loop(0, nchunk, zchunk, 0)
    for u in range(4):
        pltpu.sync_copy(dband.at[u],
                        den_sp.at[pl.ds(pl.multiple_of(s * 2560 + u * 640, 128), 640)])
    plsc.subcore_barrier()

    lanes = lax.broadcasted_iota(jnp.int32, (16,), 0)

    def p1_step(j, mv, IS, ID, IG, IX, AL, semg, semi, semw):
        base = ebase + j * B
        pltpu.make_async_copy(src_hbm.at[pl.ds(0, B)], IS, semi).wait()
        pltpu.make_async_copy(src_hbm.at[pl.ds(0, B)], ID, semi).wait()
        for t in range(5):
            IG[pl.ds(16 * t, 16)] = IS[pl.ds(16 * t, 16)] + cbase
            IX[pl.ds(16 * t, 16)] = ID[pl.ds(16 * t, 16)] + cbase
        cp1 = [
            pltpu.async_copy(xl_hbm.at[IG.at[pl.ds(0, 32)]], xlb.at[pl.ds(0, 32)], sem0),
            pltpu.async_copy(xr_hbm.at[IX.at[pl.ds(0, 32)]], xrb.at[pl.ds(0, 32)], sem0),
            pltpu.async_copy(ee_hbm.at[pl.ds(c * E + base, 32)], eeb.at[pl.ds(0, 32)], sem0),
        ]
        cp2 = [
            pltpu.async_copy(xl_hbm.at[IG.at[pl.ds(32, 48)]], xlb.at[pl.ds(32, 48)], sem1),
            pltpu.async_copy(xr_hbm.at[IX.at[pl.ds(32, 48)]], xrb.at[pl.ds(32, 48)], sem1),
            pltpu.async_copy(ee_hbm.at[pl.ds(c * E + base + 32, 48)], eeb.at[pl.ds(32, 48)], sem1),
        ]
        b2 = ebase + jnp.minimum(j + 2, NBLK - 1) * B
        pltpu.async_copy(src_hbm.at[pl.ds(b2, B)], IS, semi)
        pltpu.async_copy(dst_hbm.at[pl.ds(b2, B)], ID, semi)
        for cp in cp1:
            cp.wait()

        @pl.when(j >= 2)
        def _():
            pltpu.make_async_copy(al_hbm.at[0].at[0].at[0], AL, semw).wait()

        for t in range(5):
            if t == 2:
                for cp in cp2:
                    cp.wait()
            def pedge(i, carry):
                e = 16 * t + i
                outs = []
                for k in range(4):
                    z0 = xlb[e, pl.ds(32 * k, 16)] + xrb[e, pl.ds(32 * k, 16)] + eeb[e, pl.ds(32 * k, 16)]
                    z1 = xlb[e, pl.ds(32 * k + 16, 16)] + xrb[e, pl.ds(32 * k + 16, 16)] + eeb[e, pl.ds(32 * k + 16, 16)]
                    h0 = jnp.maximum(z0, 0.2 * z0)
                    h1 = jnp.maximum(z1, 0.2 * z1)
                    ts = h0 * attb[pl.ds(32 * k, 16)] + h1 * attb[pl.ds(32 * k + 16, 16)]
                    for st in (1, 2, 4, 8):
                        ts = ts + ts.at[lanes ^ st].get(mode="promise_in_bounds")
                    outs.append(jnp.where(lanes == i, ts, carry[k]))
                return tuple(outs)
            a = lax.fori_loop(0, 16, pedge, (zv, zv, zv, zv))
            for k in range(4):
                AL[k, pl.ds(16 * t, 16)] = a[k]
                mv = jnp.maximum(mv, a[k])
        pltpu.async_copy(AL, al_hbm.at[c].at[s].at[j], semw)
        return mv

    pA = (isrc, idst, igs, ixr, alb2, sem0, sem2, sem4)
    pB = (idx0, idx1, idx2, idx3, alb2b, sem1, sem3, sem5)
    pltpu.async_copy(src_hbm.at[pl.ds(ebase, B)], isrc, sem2)
    pltpu.async_copy(dst_hbm.at[pl.ds(ebase, B)], idst, sem2)
    pltpu.async_copy(src_hbm.at[pl.ds(ebase + B, B)], idx0, sem3)
    pltpu.async_copy(dst_hbm.at[pl.ds(ebase + B, B)], idx1, sem3)

    def p1_pair(m, mv):
        mv = p1_step(2 * m, mv, *pA)
        mv = p1_step(2 * m + 1, mv, *pB)
        return mv
    mv = lax.fori_loop(0, (NBLK - 1) // 2, p1_pair,
                       jnp.full((16,), NEG, jnp.float32))
    mv = p1_step(NBLK - 1, mv, *pA)
    pltpu.make_async_copy(al_hbm.at[0].at[0].at[0], alb2, sem4).wait()
    pltpu.make_async_copy(al_hbm.at[0].at[0].at[0], alb2b, sem5).wait()
    for IS, ID, semi in ((isrc, idst, sem2), (idx0, idx1, sem3)):
        pltpu.make_async_copy(src_hbm.at[pl.ds(0, B)], IS, semi).wait()
        pltpu.make_async_copy(src_hbm.at[pl.ds(0, B)], ID, semi).wait()

    for m in range(8):
        mxst[pl.ds(16 * m, 16)] = mv
    pltpu.sync_copy(mxst, maxstage.at[s])
    plsc.subcore_barrier()
    pltpu.sync_copy(maxstage, mxrd)
    gv = mxrd[0, pl.ds(0, 16)]
    for t in range(1, 16):
        gv = jnp.maximum(gv, mxrd[t, pl.ds(0, 16)])
    for st in (1, 2, 4, 8):
        gv = jnp.maximum(gv, gv.at[lanes ^ st].get(mode="promise_in_bounds"))
    G = gv

    NP = 10240

    def p2_start(j, X, IS, IG, ID, AL, gsem, asem):
        base = ebase + j * B
        pltpu.sync_copy(src_hbm.at[pl.ds(base, B)], IS)
        pltpu.sync_copy(dst_hbm.at[pl.ds(base, B)], ID)
        for t in range(5):
            IG[pl.ds(16 * t, 16)] = IS[pl.ds(16 * t, 16)] + cbase
        pltpu.async_copy(xl_hbm.at[IG], X, gsem)
        pltpu.async_copy(al_hbm.at[c].at[s].at[j], AL, asem)

    def p2_finish(X, ID, AL, gsem, asem):
        pltpu.make_async_copy(al_hbm.at[0].at[0].at[0], AL, asem).wait()
        for k in range(4):
            for t in range(5):
                exb4[k][pl.ds(16 * t, 16)] = jnp.exp(AL[k, pl.ds(16 * t, 16)] - G)
        for k in range(4):
            for t in range(5):
                idx2[pl.ds(16 * t, 16)] = ID[pl.ds(16 * t, 16)] + (NP * k)
            pltpu.sync_copy(exb4[k], den_sp.at[idx2], add=True)
        pltpu.make_async_copy(xl_hbm.at[pl.ds(0, B)], X, gsem).wait()
        for t in range(5):
            a16 = [exb4[k][pl.ds(16 * t, 16)] for k in range(4)]

            def pedge(i, _):
                e = 16 * t + i
                sel = lanes * 0 + i
                for k in range(4):
                    av = a16[k].at[sel].get(mode="promise_in_bounds")
                    X[e, pl.ds(32 * k, 16)] = X[e, pl.ds(32 * k, 16)] * av
                    X[e, pl.ds(32 * k + 16, 16)] = X[e, pl.ds(32 * k + 16, 16)] * av
                return 0
            lax.fori_loop(0, 16, pedge, 0)
        pltpu.sync_copy(X, acc.at[ID], add=True)

    setA = (xlb, isrc, igs, idst, alb2, sem0, sem2)
    setB = (eeb, ixr, idx0, idx1, alb2b, sem1, sem3)

    def _start(j, S):
        p2_start(j, S[0], S[1], S[2], S[3], S[4], S[5], S[6])

    def _finish(S):
        p2_finish(S[0], S[3], S[4], S[5], S[6])

    _start(0, setA)

    def p2_pair(m, _):
        _start(2 * m + 1, setB)
        _finish(setA)
        _start(2 * m + 2, setA)
        _finish(setB)
        return 0
    lax.fori_loop(0, (NBLK - 1) // 2, p2_pair, 0)
    _finish(setA)
    plsc.subcore_barrier()

    for k in range(4):
        pltpu.sync_copy(den_sp.at[pl.ds(pl.multiple_of(NP * k + s * 640, 128), 640)],
                        dband.at[k])
    for k in range(4):
        for t in range(40):
            dband[k, pl.ds(16 * t, 16)] = 1.0 / (dband[k, pl.ds(16 * t, 16)] + 1e-16)
    nv = jnp.where(s < 15, 8, 5)
    for v in range(8):
        @pl.when(v < nv)
        def _():
            pltpu.sync_copy(
                acc.at[pl.ds(pl.multiple_of(s * 640 + v * 80, 8), 80)], xlb)
            for g in range(5):
                rb = [dband[k, pl.ds(16 * (5 * v + g), 16)] for k in range(4)]

                def prow(i, _):
                    e = 16 * g + i
                    sel = lanes * 0 + i
                    for k in range(4):
                        av = rb[k].at[sel].get(mode="promise_in_bounds")
                        xlb[e, pl.ds(32 * k, 16)] = xlb[e, pl.ds(32 * k, 16)] * av
                        xlb[e, pl.ds(32 * k + 16, 16)] = xlb[e, pl.ds(32 * k + 16, 16)] * av
                    return 0
                lax.fori_loop(0, 16, prow, 0)
            pltpu.sync_copy(
                xlb, out_hbm.at[pl.ds(pl.multiple_of(cbase + s * 640 + v * 80, 8), 80)])


@jax.jit
def _sc_edge(xl_cat, xr_cat, ee_cat, src, dst, att_flat):
    mesh = plsc.VectorSubcoreMesh(core_axis_name="c", subcore_axis_name="s")
    f = pl.kernel(
        _sc_body,
        out_type=[
            jax.ShapeDtypeStruct((2 * N, HALF), jnp.float32),
            jax.ShapeDtypeStruct((2, 16, NBLK, 4, B), jnp.float32),
        ],
        mesh=mesh,
        scratch_types=(
            [pltpu.VMEM((B,), jnp.int32)] * 4
            + [pltpu.VMEM((B,), jnp.int32)] * 4
            + [pltpu.VMEM((B,), jnp.float32)] * 4
            + [
                pltpu.VMEM((B, HALF), jnp.float32),
                pltpu.VMEM((B, HALF), jnp.float32),
                pltpu.VMEM((B, HALF), jnp.float32),
                pltpu.VMEM((4, B), jnp.float32),
                pltpu.VMEM((4, B), jnp.float32),
                pltpu.VMEM((4, 640), jnp.float32),
                pltpu.VMEM((HALF,), jnp.float32),
                pltpu.VMEM((HALF,), jnp.float32),
                pltpu.VMEM((16, HALF), jnp.float32),
                pltpu.VMEM_SHARED((N, HALF), jnp.float32),
                pltpu.VMEM_SHARED((40960,), jnp.float32),
                pltpu.VMEM_SHARED((16, HALF), jnp.float32),
                pltpu.SemaphoreType.DMA,
                pltpu.SemaphoreType.DMA,
                pltpu.SemaphoreType.DMA,
                pltpu.SemaphoreType.DMA,
                pltpu.SemaphoreType.DMA,
                pltpu.SemaphoreType.DMA,
            ]
        ),
    )
    return f(xl_cat, xr_cat, ee_cat, src, dst, att_flat)[0]


def _proj_body(x_ref, w_ref, b_ref, o_ref):
    y = jnp.dot(x_ref[...], w_ref[0], preferred_element_type=jnp.float32)
    y = y + b_ref[0, 0, :]
    o_ref[0] = jnp.where(y > 0, y, jnp.exp(jnp.minimum(y, 0.0)) - 1.0)


def _proj(x, w, b):
    m, k = x.shape
    blk = 2000
    return pl.pallas_call(
        _proj_body,
        grid=(m // blk, 2),
        in_specs=[
            pl.BlockSpec((blk, k), lambda i, h: (i, 0)),
            pl.BlockSpec((1, k, HALF), lambda i, h: (h, 0, 0)),
            pl.BlockSpec((1, 1, HALF), lambda i, h: (h, 0, 0)),
        ],
        out_specs=pl.BlockSpec((1, blk, HALF), lambda i, h: (h, i, 0)),
        out_shape=jax.ShapeDtypeStruct((2, m, HALF), jnp.float32),
    )(x, jnp.transpose(w.reshape(k, 2, HALF), (1, 0, 2)), b.reshape(2, 1, HALF))


def _lin_body(x_ref, w_ref, b_ref, o_ref):
    y = jnp.dot(x_ref[0], w_ref[0, 0], preferred_element_type=jnp.float32)
    y = y + jnp.dot(x_ref[1], w_ref[0, 1], preferred_element_type=jnp.float32)
    o_ref[0] = y + b_ref[0, 0, :]


def _lin_split(x2, w, b):
    m = x2.shape[1]
    blk = 2000
    return pl.pallas_call(
        _lin_body,
        grid=(m // blk, 2),
        in_specs=[
            pl.BlockSpec((2, blk, HALF), lambda i, h: (0, i, 0)),
            pl.BlockSpec((1, 2, HALF, HALF), lambda i, h: (h, 0, 0, 0)),
            pl.BlockSpec((1, 1, HALF), lambda i, h: (h, 0, 0)),
        ],
        out_specs=pl.BlockSpec((1, blk, HALF), lambda i, h: (h, i, 0)),
        out_shape=jax.ShapeDtypeStruct((2, m, HALF), jnp.float32),
    )(x2, jnp.transpose(w.reshape(2, HALF, 2, HALF), (2, 0, 1, 3)),
      b.reshape(2, 1, HALF))


def _ee_body(ea_ref, we_ref, o_ref):
    ea = ea_ref[...]
    y = ea[:, 0][:, None] * we_ref[0, 0, :][None, :]
    for j in range(1, 4):
        y = y + ea[:, j][:, None] * we_ref[0, j, :][None, :]
    o_ref[0] = y


def _ee(ea, we):
    m = ea.shape[0]
    blk = 2000
    return pl.pallas_call(
        _ee_body,
        grid=(m // blk, 2),
        in_specs=[
            pl.BlockSpec((blk, 4), lambda i, h: (i, 0)),
            pl.BlockSpec((1, 4, HALF), lambda i, h: (h, 0, 0)),
        ],
        out_specs=pl.BlockSpec((1, blk, HALF), lambda i, h: (h, i, 0)),
        out_shape=jax.ShapeDtypeStruct((2, m, HALF), jnp.float32),
    )(ea, jnp.transpose(we.reshape(4, 2, HALF), (1, 0, 2)))


def _bn1_body(o1_ref, bias_ref, g_ref, b_ref, o_ref):
    x = o1_ref[0] + bias_ref[0, 0, :]
    y = jnp.where(x > 0, x, jnp.exp(jnp.minimum(x, 0.0)) - 1.0)
    m = jnp.mean(y, axis=0)
    v = jnp.mean((y - m[None, :]) ** 2, axis=0)
    o_ref[0] = (y - m[None, :]) * jax.lax.rsqrt(v + 1e-5)[None, :] * g_ref[0, 0, :] + b_ref[0, 0, :]


def _bn2_body(o1_ref, o2_ref, bias_ref, g_ref, b_ref, o_ref):
    x = o1_ref[0] + o2_ref[0] + bias_ref[0, 0, :]
    y = jnp.where(x > 0, x, jnp.exp(jnp.minimum(x, 0.0)) - 1.0)
    m = jnp.mean(y, axis=0)
    v = jnp.mean((y - m[None, :]) ** 2, axis=0)
    o_ref[0] = (y - m[None, :]) * jax.lax.rsqrt(v + 1e-5)[None, :] * g_ref[0, 0, :] + b_ref[0, 0, :]


def _bn(o1, o2, bias, g, b):
    vec = pl.BlockSpec((1, 1, HALF), lambda h: (h, 0, 0))
    full = pl.BlockSpec((1, N, HALF), lambda h: (h, 0, 0))
    if o2 is None:
        return pl.pallas_call(
            _bn1_body, grid=(2,),
            in_specs=[full, vec, vec, vec],
            out_specs=full,
            out_shape=jax.ShapeDtypeStruct((2, N, HALF), jnp.float32),
        )(o1, bias.reshape(2, 1, HALF), g.reshape(2, 1, HALF), b.reshape(2, 1, HALF))
    return pl.pallas_call(
        _bn2_body, grid=(2,),
        in_specs=[full, full, vec, vec, vec],
        out_specs=full,
        out_shape=jax.ShapeDtypeStruct((2, N, HALF), jnp.float32),
    )(o1, o2, bias.reshape(2, 1, HALF), g.reshape(2, 1, HALF), b.reshape(2, 1, HALF))


def _head_body(x_ref, w_ref, b_ref, o_ref):
    y = jnp.dot(x_ref[0], w_ref[0][:, None], preferred_element_type=jnp.float32)
    y = y + jnp.dot(x_ref[1], w_ref[1][:, None], preferred_element_type=jnp.float32)
    o_ref[...] = y + b_ref[0]


def _head(x2, w, b):
    blk = 2000
    return pl.pallas_call(
        _head_body,
        grid=(N // blk,),
        in_specs=[
            pl.BlockSpec((2, blk, HALF), lambda i: (0, i, 0)),
            pl.BlockSpec((2, HALF), lambda i: (0, 0)),
            pl.BlockSpec((1,), lambda i: (0,)),
        ],
        out_specs=pl.BlockSpec((blk, 1), lambda i: (i, 0)),
        out_shape=jax.ShapeDtypeStruct((N, 1), jnp.float32),
    )(x2, w.reshape(2, HALF), b)


def _gatv2_sc(p, x_src2, x_dst2, ei, ea):
    xl = _lin_split(x_src2, p["wl"], p["bl"])
    xr = _lin_split(x_dst2, p["wr"], p["br"])
    ee = _ee(ea, p["we"])
    o = _sc_edge(xl.reshape(2 * N, HALF), xr.reshape(2 * N, HALF),
                 ee.reshape(2 * E, HALF), ei[0], ei[1],
                 p["att"].reshape(HID))
    return o.reshape(2, N, HALF)


def kernel(x_lidar, x_radar1, x_radar2, ei_ll, ei_r1r1, ei_r2r2, ei_lr1, ei_lr2,
           ea_ll, ea_r1r1, ea_r2r2, ea_lr1, ea_lr2, params):
    x = {
        "lidar": _proj(x_lidar, params["proj"]["lidar"]["w"], params["proj"]["lidar"]["b"]),
        "radar1": _proj(x_radar1, params["proj"]["radar1"]["w"], params["proj"]["radar1"]["b"]),
        "radar2": _proj(x_radar2, params["proj"]["radar2"]["w"], params["proj"]["radar2"]["b"]),
    }
    for layer in params["layers"]:
        c = layer["conv"]
        o_ll = _gatv2_sc(c["ll"], x["lidar"], x["lidar"], ei_ll, ea_ll)
        o_r1r1 = _gatv2_sc(c["r1r1"], x["radar1"], x["radar1"], ei_r1r1, ea_r1r1)
        o_lr1 = _gatv2_sc(c["lr1"], x["lidar"], x["radar1"], ei_lr1, ea_lr1)
        o_r2r2 = _gatv2_sc(c["r2r2"], x["radar2"], x["radar2"], ei_r2r2, ea_r2r2)
        o_lr2 = _gatv2_sc(c["lr2"], x["lidar"], x["radar2"], ei_lr2, ea_lr2)
        bn = layer["bn"]
        x = {
            "lidar": _bn(o_ll, None, c["ll"]["bias"], bn["lidar"]["g"], bn["lidar"]["b"]),
            "radar1": _bn(o_r1r1, o_lr1, c["r1r1"]["bias"] + c["lr1"]["bias"],
                          bn["radar1"]["g"], bn["radar1"]["b"]),
            "radar2": _bn(o_r2r2, o_lr2, c["r2r2"]["bias"] + c["lr2"]["bias"],
                          bn["radar2"]["g"], bn["radar2"]["b"]),
        }
    out_l = _head(x["lidar"], params["head_lidar"]["w"], params["head_lidar"]["b"])
    out_r1 = _head(x["radar1"], params["head_radar"]["w"], params["head_radar"]["b"])
    out_r2 = _head(x["radar2"], params["head_radar"]["w"], params["head_radar"]["b"])
    return (out_l, out_r1, out_r2)


# speedup vs baseline: 30.1695x; 1.0813x over previous
import functools
import jax
import jax.numpy as jnp
from jax import lax
from jax.experimental import pallas as pl
from jax.experimental.pallas import tpu as pltpu
from jax.experimental.pallas import tpu_sc as plsc

N = 10000
E = 160000
HID = 256
HALF = 128
B = 80
EPT = E // 16
NBLK = EPT // B
NEG = -1e30


def _sc_body(xl_hbm, xr_hbm, ee_hbm, src_hbm, dst_hbm, att_hbm, out_hbm, al_hbm,
             isrc, idst, igs, ixr, idx0, idx1, idx2, idx3,
             exb0, exb1, exb2, exb3,
             xlb, xrb, eeb, alb2, alb2b, dband, attb, mxst, mxrd,
             acc, den_sp, maxstage, sem0, sem1, sem2, sem3, sem4, sem5):
    exb4 = [exb0, exb1, exb2, exb3]
    c = lax.axis_index("c")
    s = lax.axis_index("s")
    cbase = c * N
    ebase = s * EPT
    zv = jnp.zeros((16,), jnp.float32)

    def zrow(i, _):
        for m in range(8):
            xlb[i, pl.ds(16 * m, 16)] = zv
        return 0
    lax.fori_loop(0, B, zrow, 0)
    for u in range(4):
        def zdb(i, _):
            dband[u, pl.ds(i * 16, 16)] = zv
            return 0
        lax.fori_loop(0, 40, zdb, 0)
    pltpu.sync_copy(att_hbm.at[pl.ds(c * HALF, HALF)], attb)

    nchunk = jnp.where(s < 15, 8, 5)
    def zchunk(k, _):
        pltpu.sync_copy(xlb, acc.at[pl.ds(pl.multiple_of(s * 640 + k * 80, 8), 80)])
        return 0
    lax.fori_loop(0, nchunk, zchunk, 0)
    for u in range(4):
        pltpu.sync_copy(dband.at[u],
                        den_sp.at[pl.ds(pl.multiple_of(s * 2560 + u * 640, 128), 640)])
    plsc.subcore_barrier()

    lanes = lax.broadcasted_iota(jnp.int32, (16,), 0)

    def p1_step(j, mv, IS, ID, IG, IX, AL, semg, semi, semw):
        base = ebase + j * B
        pltpu.make_async_copy(src_hbm.at[pl.ds(0, B)], IS, semi).wait()
        pltpu.make_async_copy(src_hbm.at[pl.ds(0, B)], ID, semi).wait()
        for t in range(5):
            IG[pl.ds(16 * t, 16)] = IS[pl.ds(16 * t, 16)] + cbase
            IX[pl.ds(16 * t, 16)] = ID[pl.ds(16 * t, 16)] + cbase
        cp1 = [
            pltpu.async_copy(xl_hbm.at[IG.at[pl.ds(0, 32)]], xlb.at[pl.ds(0, 32)], sem0),
            pltpu.async_copy(xr_hbm.at[IX.at[pl.ds(0, 32)]], xrb.at[pl.ds(0, 32)], sem0),
            pltpu.async_copy(ee_hbm.at[pl.ds(c * E + base, 32)], eeb.at[pl.ds(0, 32)], sem0),
        ]
        cp2 = [
            pltpu.async_copy(xl_hbm.at[IG.at[pl.ds(32, 48)]], xlb.at[pl.ds(32, 48)], sem1),
            pltpu.async_copy(xr_hbm.at[IX.at[pl.ds(32, 48)]], xrb.at[pl.ds(32, 48)], sem1),
            pltpu.async_copy(ee_hbm.at[pl.ds(c * E + base + 32, 48)], eeb.at[pl.ds(32, 48)], sem1),
        ]
        b2 = ebase + jnp.minimum(j + 2, NBLK - 1) * B
        pltpu.async_copy(src_hbm.at[pl.ds(b2, B)], IS, semi)
        pltpu.async_copy(dst_hbm.at[pl.ds(b2, B)], ID, semi)
        for cp in cp1:
            cp.wait()

        @pl.when(j >= 2)
        def _():
            pltpu.make_async_copy(al_hbm.at[0].at[0].at[0], AL, semw).wait()

        for t in range(5):
            if t == 2:
                for cp in cp2:
                    cp.wait()
            def pedge(i, carry):
                e = 16 * t + i
                outs = []
                for k in range(4):
                    z0 = xlb[e, pl.ds(32 * k, 16)] + xrb[e, pl.ds(32 * k, 16)] + eeb[e, pl.ds(32 * k, 16)]
                    z1 = xlb[e, pl.ds(32 * k + 16, 16)] + xrb[e, pl.ds(32 * k + 16, 16)] + eeb[e, pl.ds(32 * k + 16, 16)]
                    h0 = jnp.maximum(z0, 0.2 * z0)
                    h1 = jnp.maximum(z1, 0.2 * z1)
                    ts = h0 * attb[pl.ds(32 * k, 16)] + h1 * attb[pl.ds(32 * k + 16, 16)]
                    for st in (1, 2, 4, 8):
                        ts = ts + ts.at[lanes ^ st].get(mode="promise_in_bounds")
                    outs.append(jnp.where(lanes == i, ts, carry[k]))
                return tuple(outs)
            a = lax.fori_loop(0, 16, pedge, (zv, zv, zv, zv))
            for k in range(4):
                AL[k, pl.ds(16 * t, 16)] = a[k]
                mv = jnp.maximum(mv, a[k])
        pltpu.async_copy(AL, al_hbm.at[c].at[s].at[j], semw)
        return mv

    pA = (isrc, idst, igs, ixr, alb2, sem0, sem2, sem4)
    pB = (idx0, idx1, idx2, idx3, alb2b, sem1, sem3, sem5)
    pltpu.async_copy(src_hbm.at[pl.ds(ebase, B)], isrc, sem2)
    pltpu.async_copy(dst_hbm.at[pl.ds(ebase, B)], idst, sem2)
    pltpu.async_copy(src_hbm.at[pl.ds(ebase + B, B)], idx0, sem3)
    pltpu.async_copy(dst_hbm.at[pl.ds(ebase + B, B)], idx1, sem3)

    def p1_pair(m, mv):
        mv = p1_step(2 * m, mv, *pA)
        mv = p1_step(2 * m + 1, mv, *pB)
        return mv
    mv = lax.fori_loop(0, (NBLK - 1) // 2, p1_pair,
                       jnp.full((16,), NEG, jnp.float32))
    mv = p1_step(NBLK - 1, mv, *pA)
    pltpu.make_async_copy(al_hbm.at[0].at[0].at[0], alb2, sem4).wait()
    pltpu.make_async_copy(al_hbm.at[0].at[0].at[0], alb2b, sem5).wait()
    for IS, ID, semi in ((isrc, idst, sem2), (idx0, idx1, sem3)):
        pltpu.make_async_copy(src_hbm.at[pl.ds(0, B)], IS, semi).wait()
        pltpu.make_async_copy(src_hbm.at[pl.ds(0, B)], ID, semi).wait()

    for m in range(8):
        mxst[pl.ds(16 * m, 16)] = mv
    pltpu.sync_copy(mxst, maxstage.at[s])
    plsc.subcore_barrier()
    pltpu.sync_copy(maxstage, mxrd)
    gv = mxrd[0, pl.ds(0, 16)]
    for t in range(1, 16):
        gv = jnp.maximum(gv, mxrd[t, pl.ds(0, 16)])
    for st in (1, 2, 4, 8):
        gv = jnp.maximum(gv, gv.at[lanes ^ st].get(mode="promise_in_bounds"))
    G = gv

    NP = 10240

    def p2_start(j, X, IS, IG, ID, AL, gsem, asem, isem):
        base = ebase + j * B
        pltpu.make_async_copy(src_hbm.at[pl.ds(0, B)], IS, isem).wait()
        pltpu.sync_copy(dst_hbm.at[pl.ds(base, B)], ID)
        for t in range(5):
            IG[pl.ds(16 * t, 16)] = IS[pl.ds(16 * t, 16)] + cbase
        pltpu.async_copy(xl_hbm.at[IG], X, gsem)
        pltpu.async_copy(al_hbm.at[c].at[s].at[j], AL, asem)
        b2 = ebase + jnp.minimum(j + 2, NBLK - 1) * B
        pltpu.async_copy(src_hbm.at[pl.ds(b2, B)], IS, isem)

    def p2_finish(X, ID, AL, gsem, asem):
        pltpu.make_async_copy(al_hbm.at[0].at[0].at[0], AL, asem).wait()
        for k in range(4):
            for t in range(5):
                exb4[k][pl.ds(16 * t, 16)] = jnp.exp(AL[k, pl.ds(16 * t, 16)] - G)
        for k in range(4):
            for t in range(5):
                idx2[pl.ds(16 * t, 16)] = ID[pl.ds(16 * t, 16)] + (NP * k)
            pltpu.sync_copy(exb4[k], den_sp.at[idx2], add=True)
        pltpu.make_async_copy(xl_hbm.at[pl.ds(0, B)], X, gsem).wait()
        for t in range(5):
            a16 = [exb4[k][pl.ds(16 * t, 16)] for k in range(4)]

            def pedge(i, _):
                e = 16 * t + i
                sel = lanes * 0 + i
                for k in range(4):
                    av = a16[k].at[sel].get(mode="promise_in_bounds")
                    X[e, pl.ds(32 * k, 16)] = X[e, pl.ds(32 * k, 16)] * av
                    X[e, pl.ds(32 * k + 16, 16)] = X[e, pl.ds(32 * k + 16, 16)] * av
                return 0
            lax.fori_loop(0, 16, pedge, 0)
        pltpu.sync_copy(X, acc.at[ID], add=True)

    setA = (xlb, isrc, igs, idst, alb2, sem0, sem2, sem4)
    setB = (eeb, ixr, idx0, idx1, alb2b, sem1, sem3, sem5)

    def _start(j, S):
        p2_start(j, *S)

    def _finish(S):
        p2_finish(S[0], S[3], S[4], S[5], S[6])

    pltpu.async_copy(src_hbm.at[pl.ds(ebase, B)], isrc, sem4)
    pltpu.async_copy(src_hbm.at[pl.ds(ebase + B, B)], ixr, sem5)
    _start(0, setA)

    def p2_pair(m, _):
        _start(2 * m + 1, setB)
        _finish(setA)
        _start(2 * m + 2, setA)
        _finish(setB)
        return 0
    lax.fori_loop(0, (NBLK - 1) // 2, p2_pair, 0)
    _finish(setA)
    pltpu.make_async_copy(src_hbm.at[pl.ds(0, B)], isrc, sem4).wait()
    pltpu.make_async_copy(src_hbm.at[pl.ds(0, B)], ixr, sem5).wait()
    plsc.subcore_barrier()

    for k in range(4):
        pltpu.sync_copy(den_sp.at[pl.ds(pl.multiple_of(NP * k + s * 640, 128), 640)],
                        dband.at[k])
    for k in range(4):
        for t in range(40):
            dband[k, pl.ds(16 * t, 16)] = 1.0 / (dband[k, pl.ds(16 * t, 16)] + 1e-16)
    nv = jnp.where(s < 15, 8, 5)
    for v in range(8):
        @pl.when(v < nv)
        def _():
            pltpu.sync_copy(
                acc.at[pl.ds(pl.multiple_of(s * 640 + v * 80, 8), 80)], xlb)
            for g in range(5):
                rb = [dband[k, pl.ds(16 * (5 * v + g), 16)] for k in range(4)]

                def prow(i, _):
                    e = 16 * g + i
                    sel = lanes * 0 + i
                    for k in range(4):
                        av = rb[k].at[sel].get(mode="promise_in_bounds")
                        xlb[e, pl.ds(32 * k, 16)] = xlb[e, pl.ds(32 * k, 16)] * av
                        xlb[e, pl.ds(32 * k + 16, 16)] = xlb[e, pl.ds(32 * k + 16, 16)] * av
                    return 0
                lax.fori_loop(0, 16, prow, 0)
            pltpu.sync_copy(
                xlb, out_hbm.at[pl.ds(pl.multiple_of(cbase + s * 640 + v * 80, 8), 80)])


@jax.jit
def _sc_edge(xl_cat, xr_cat, ee_cat, src, dst, att_flat):
    mesh = plsc.VectorSubcoreMesh(core_axis_name="c", subcore_axis_name="s")
    f = pl.kernel(
        _sc_body,
        out_type=[
            jax.ShapeDtypeStruct((2 * N, HALF), jnp.float32),
            jax.ShapeDtypeStruct((2, 16, NBLK, 4, B), jnp.float32),
        ],
        mesh=mesh,
        scratch_types=(
            [pltpu.VMEM((B,), jnp.int32)] * 4
            + [pltpu.VMEM((B,), jnp.int32)] * 4
            + [pltpu.VMEM((B,), jnp.float32)] * 4
            + [
                pltpu.VMEM((B, HALF), jnp.float32),
                pltpu.VMEM((B, HALF), jnp.float32),
                pltpu.VMEM((B, HALF), jnp.float32),
                pltpu.VMEM((4, B), jnp.float32),
                pltpu.VMEM((4, B), jnp.float32),
                pltpu.VMEM((4, 640), jnp.float32),
                pltpu.VMEM((HALF,), jnp.float32),
                pltpu.VMEM((HALF,), jnp.float32),
                pltpu.VMEM((16, HALF), jnp.float32),
                pltpu.VMEM_SHARED((N, HALF), jnp.float32),
                pltpu.VMEM_SHARED((40960,), jnp.float32),
                pltpu.VMEM_SHARED((16, HALF), jnp.float32),
                pltpu.SemaphoreType.DMA,
                pltpu.SemaphoreType.DMA,
                pltpu.SemaphoreType.DMA,
                pltpu.SemaphoreType.DMA,
                pltpu.SemaphoreType.DMA,
                pltpu.SemaphoreType.DMA,
            ]
        ),
    )
    return f(xl_cat, xr_cat, ee_cat, src, dst, att_flat)[0]


def _proj_body(x_ref, w_ref, b_ref, o_ref):
    y = jnp.dot(x_ref[...], w_ref[0], preferred_element_type=jnp.float32)
    y = y + b_ref[0, 0, :]
    o_ref[0] = jnp.where(y > 0, y, jnp.exp(jnp.minimum(y, 0.0)) - 1.0)


def _proj(x, w, b):
    m, k = x.shape
    blk = 2000
    return pl.pallas_call(
        _proj_body,
        grid=(m // blk, 2),
        in_specs=[
            pl.BlockSpec((blk, k), lambda i, h: (i, 0)),
            pl.BlockSpec((1, k, HALF), lambda i, h: (h, 0, 0)),
            pl.BlockSpec((1, 1, HALF), lambda i, h: (h, 0, 0)),
        ],
        out_specs=pl.BlockSpec((1, blk, HALF), lambda i, h: (h, i, 0)),
        out_shape=jax.ShapeDtypeStruct((2, m, HALF), jnp.float32),
    )(x, jnp.transpose(w.reshape(k, 2, HALF), (1, 0, 2)), b.reshape(2, 1, HALF))


def _lin_body(x_ref, w_ref, b_ref, o_ref):
    y = jnp.dot(x_ref[0], w_ref[0, 0], preferred_element_type=jnp.float32)
    y = y + jnp.dot(x_ref[1], w_ref[0, 1], preferred_element_type=jnp.float32)
    o_ref[0] = y + b_ref[0, 0, :]


def _lin_split(x2, w, b):
    m = x2.shape[1]
    blk = 2000
    return pl.pallas_call(
        _lin_body,
        grid=(m // blk, 2),
        in_specs=[
            pl.BlockSpec((2, blk, HALF), lambda i, h: (0, i, 0)),
            pl.BlockSpec((1, 2, HALF, HALF), lambda i, h: (h, 0, 0, 0)),
            pl.BlockSpec((1, 1, HALF), lambda i, h: (h, 0, 0)),
        ],
        out_specs=pl.BlockSpec((1, blk, HALF), lambda i, h: (h, i, 0)),
        out_shape=jax.ShapeDtypeStruct((2, m, HALF), jnp.float32),
    )(x2, jnp.transpose(w.reshape(2, HALF, 2, HALF), (2, 0, 1, 3)),
      b.reshape(2, 1, HALF))


def _ee_body(ea_ref, we_ref, o_ref):
    ea = ea_ref[...]
    y = ea[:, 0][:, None] * we_ref[0, 0, :][None, :]
    for j in range(1, 4):
        y = y + ea[:, j][:, None] * we_ref[0, j, :][None, :]
    o_ref[0] = y


def _ee(ea, we):
    m = ea.shape[0]
    blk = 2000
    return pl.pallas_call(
        _ee_body,
        grid=(m // blk, 2),
        in_specs=[
            pl.BlockSpec((blk, 4), lambda i, h: (i, 0)),
            pl.BlockSpec((1, 4, HALF), lambda i, h: (h, 0, 0)),
        ],
        out_specs=pl.BlockSpec((1, blk, HALF), lambda i, h: (h, i, 0)),
        out_shape=jax.ShapeDtypeStruct((2, m, HALF), jnp.float32),
    )(ea, jnp.transpose(we.reshape(4, 2, HALF), (1, 0, 2)))


def _bn1_body(o1_ref, bias_ref, g_ref, b_ref, o_ref):
    x = o1_ref[0] + bias_ref[0, 0, :]
    y = jnp.where(x > 0, x, jnp.exp(jnp.minimum(x, 0.0)) - 1.0)
    m = jnp.mean(y, axis=0)
    v = jnp.mean((y - m[None, :]) ** 2, axis=0)
    o_ref[0] = (y - m[None, :]) * jax.lax.rsqrt(v + 1e-5)[None, :] * g_ref[0, 0, :] + b_ref[0, 0, :]


def _bn2_body(o1_ref, o2_ref, bias_ref, g_ref, b_ref, o_ref):
    x = o1_ref[0] + o2_ref[0] + bias_ref[0, 0, :]
    y = jnp.where(x > 0, x, jnp.exp(jnp.minimum(x, 0.0)) - 1.0)
    m = jnp.mean(y, axis=0)
    v = jnp.mean((y - m[None, :]) ** 2, axis=0)
    o_ref[0] = (y - m[None, :]) * jax.lax.rsqrt(v + 1e-5)[None, :] * g_ref[0, 0, :] + b_ref[0, 0, :]


def _bn(o1, o2, bias, g, b):
    vec = pl.BlockSpec((1, 1, HALF), lambda h: (h, 0, 0))
    full = pl.BlockSpec((1, N, HALF), lambda h: (h, 0, 0))
    if o2 is None:
        return pl.pallas_call(
            _bn1_body, grid=(2,),
            in_specs=[full, vec, vec, vec],
            out_specs=full,
            out_shape=jax.ShapeDtypeStruct((2, N, HALF), jnp.float32),
        )(o1, bias.reshape(2, 1, HALF), g.reshape(2, 1, HALF), b.reshape(2, 1, HALF))
    return pl.pallas_call(
        _bn2_body, grid=(2,),
        in_specs=[full, full, vec, vec, vec],
        out_specs=full,
        out_shape=jax.ShapeDtypeStruct((2, N, HALF), jnp.float32),
    )(o1, o2, bias.reshape(2, 1, HALF), g.reshape(2, 1, HALF), b.reshape(2, 1, HALF))


def _head_body(x_ref, w_ref, b_ref, o_ref):
    y = jnp.dot(x_ref[0], w_ref[0][:, None], preferred_element_type=jnp.float32)
    y = y + jnp.dot(x_ref[1], w_ref[1][:, None], preferred_element_type=jnp.float32)
    o_ref[...] = y + b_ref[0]


def _head(x2, w, b):
    blk = 2000
    return pl.pallas_call(
        _head_body,
        grid=(N // blk,),
        in_specs=[
            pl.BlockSpec((2, blk, HALF), lambda i: (0, i, 0)),
            pl.BlockSpec((2, HALF), lambda i: (0, 0)),
            pl.BlockSpec((1,), lambda i: (0,)),
        ],
        out_specs=pl.BlockSpec((blk, 1), lambda i: (i, 0)),
        out_shape=jax.ShapeDtypeStruct((N, 1), jnp.float32),
    )(x2, w.reshape(2, HALF), b)


def _gatv2_sc(p, x_src2, x_dst2, ei, ea):
    xl = _lin_split(x_src2, p["wl"], p["bl"])
    xr = _lin_split(x_dst2, p["wr"], p["br"])
    ee = _ee(ea, p["we"])
    o = _sc_edge(xl.reshape(2 * N, HALF), xr.reshape(2 * N, HALF),
                 ee.reshape(2 * E, HALF), ei[0], ei[1],
                 p["att"].reshape(HID))
    return o.reshape(2, N, HALF)


def kernel(x_lidar, x_radar1, x_radar2, ei_ll, ei_r1r1, ei_r2r2, ei_lr1, ei_lr2,
           ea_ll, ea_r1r1, ea_r2r2, ea_lr1, ea_lr2, params):
    x = {
        "lidar": _proj(x_lidar, params["proj"]["lidar"]["w"], params["proj"]["lidar"]["b"]),
        "radar1": _proj(x_radar1, params["proj"]["radar1"]["w"], params["proj"]["radar1"]["b"]),
        "radar2": _proj(x_radar2, params["proj"]["radar2"]["w"], params["proj"]["radar2"]["b"]),
    }
    for layer in params["layers"]:
        c = layer["conv"]
        o_ll = _gatv2_sc(c["ll"], x["lidar"], x["lidar"], ei_ll, ea_ll)
        o_r1r1 = _gatv2_sc(c["r1r1"], x["radar1"], x["radar1"], ei_r1r1, ea_r1r1)
        o_lr1 = _gatv2_sc(c["lr1"], x["lidar"], x["radar1"], ei_lr1, ea_lr1)
        o_r2r2 = _gatv2_sc(c["r2r2"], x["radar2"], x["radar2"], ei_r2r2, ea_r2r2)
        o_lr2 = _gatv2_sc(c["lr2"], x["lidar"], x["radar2"], ei_lr2, ea_lr2)
        bn = layer["bn"]
        x = {
            "lidar": _bn(o_ll, None, c["ll"]["bias"], bn["lidar"]["g"], bn["lidar"]["b"]),
            "radar1": _bn(o_r1r1, o_lr1, c["r1r1"]["bias"] + c["lr1"]["bias"],
                          bn["radar1"]["g"], bn["radar1"]["b"]),
            "radar2": _bn(o_r2r2, o_lr2, c["r2r2"]["bias"] + c["lr2"]["bias"],
                          bn["radar2"]["g"], bn["radar2"]["b"]),
        }
    out_l = _head(x["lidar"], params["head_lidar"]["w"], params["head_lidar"]["b"])
    out_r1 = _head(x["radar1"], params["head_radar"]["w"], params["head_radar"]["b"])
    out_r2 = _head(x["radar2"], params["head_radar"]["w"], params["head_radar"]["b"])
    return (out_l, out_r1, out_r2)
